# Initial kernel scaffold; baseline (speedup 1.0000x reference)
import functools

import jax
import jax.numpy as jnp
from jax import lax
from jax.experimental import pallas as pl
from jax.experimental.pallas import tpu as pltpu
from jax.experimental.pallas import tpu_sc as plsc

HID = 128
N = 10000
E = 320000
NC, NS, L = 2, 16, 16
NW = NC * NS
EPT = E // NW
CH = 80
NCH = EPT // CH
RPT = N // NS
ZR = 125
NEG = -1e30


def _k1a_body(x_ref, w_ref, we_ref, att2_ref, atte_ref, xs_ref, a2_ref, v_ref):
    xs = jnp.dot(x_ref[...], w_ref[...], preferred_element_type=jnp.float32)
    xs_ref[...] = xs
    a2_ref[...] = jnp.dot(xs, att2_ref[...], preferred_element_type=jnp.float32)
    v_ref[...] = jnp.dot(we_ref[...], atte_ref[...],
                         preferred_element_type=jnp.float32)


def _k1b_body(ea_ref, v_ref, ae_ref):
    ae_ref[...] = jnp.dot(ea_ref[...], v_ref[...],
                          preferred_element_type=jnp.float32)


def _k2_body(ae_ref, asrc_ref, adst_ref, part_ref, amax_ref, als_ref):
    mean = jnp.sum(ae_ref[...]) * (1.0 / E)
    a = asrc_ref[...] + adst_ref[...] + mean
    als = jnp.where(a >= 0, a, 0.2 * a)
    als_ref[...] = als
    amax_ref[...] = jnp.maximum(jnp.max(part_ref[...], axis=0), als)


def _k3_body(part_ref, amax_ref, als_ref, r_ref, cs_ref):
    ex_s = jnp.exp(als_ref[...] - amax_ref[...])
    denom = jnp.sum(part_ref[...], axis=0) + ex_s
    r = 1.0 / (denom + 1e-16)
    r_ref[...] = r
    cs_ref[...] = ex_s * r


def _k4_body(part_ref, xs_ref, cs_ref, bias_ref, out_ref):
    acc = part_ref[0] + part_ref[1]
    out_ref[...] = acc + cs_ref[...][:, None] * xs_ref[...] + bias_ref[...]


def _leaky(x):
    return jnp.where(x >= 0, x, 0.2 * x)


def _s1_body(asrc_hbm, adst_hbm, src_hbm, dst_hbm, ae_hbm,
             alpha_out, amax_out,
             asrc_v, adst_v, src_v, dst_v, ae_v, alpha_v, amax_v):
    cid = lax.axis_index("c")
    sid = lax.axis_index("s")
    wid = sid * NC + cid
    base = wid * EPT
    pltpu.sync_copy(asrc_hbm, asrc_v)
    pltpu.sync_copy(adst_hbm, adst_v)
    pltpu.sync_copy(src_hbm.at[pl.ds(base, EPT)], src_v)
    pltpu.sync_copy(dst_hbm.at[pl.ds(base, EPT)], dst_v)
    pltpu.sync_copy(ae_hbm.at[pl.ds(base, EPT)], ae_v)

    def initb(i, c):
        amax_v[pl.ds(i * L, L)] = jnp.full((L,), NEG, jnp.float32)
        return c

    lax.fori_loop(0, N // L, initb, 0)

    def edge_block(i, c):
        sl = pl.ds(i * L, L)
        s = src_v[sl]
        d = dst_v[sl]
        a = (plsc.load_gather(asrc_v, [s]) + plsc.load_gather(adst_v, [d])
             + ae_v[sl])
        al = _leaky(a)
        alpha_v[sl] = al

        def cond(cur):
            return jnp.any(al > cur)

        def body(cur):
            plsc.store_scatter(amax_v, [d], al, mask=al > cur)
            return plsc.load_gather(amax_v, [d])

        lax.while_loop(cond, body, plsc.load_gather(amax_v, [d]))
        return c

    lax.fori_loop(0, EPT // L, edge_block, 0)
    pltpu.sync_copy(alpha_v, alpha_out.at[pl.ds(base, EPT)])
    pltpu.sync_copy(amax_v, amax_out.at[wid])


def _s2_body(dst_hbm, alpha_hbm, amax_hbm,
             ex_out, denom_out,
             amax_v, dst_v, alpha_v, ex_v, denom_v):
    cid = lax.axis_index("c")
    sid = lax.axis_index("s")
    wid = sid * NC + cid
    base = wid * EPT
    pltpu.sync_copy(amax_hbm, amax_v)
    pltpu.sync_copy(dst_hbm.at[pl.ds(base, EPT)], dst_v)
    pltpu.sync_copy(alpha_hbm.at[pl.ds(base, EPT)], alpha_v)

    def initb(i, c):
        denom_v[pl.ds(i * L, L)] = jnp.zeros((L,), jnp.float32)
        return c

    lax.fori_loop(0, N // L, initb, 0)

    def edge_block(i, c):
        sl = pl.ds(i * L, L)
        d = dst_v[sl]
        am = plsc.load_gather(amax_v, [d])
        e = jnp.exp(alpha_v[sl] - am)
        ex_v[sl] = e
        plsc.addupdate_scatter(denom_v, [d], e)
        return c

    lax.fori_loop(0, EPT // L, edge_block, 0)
    pltpu.sync_copy(ex_v, ex_out.at[pl.ds(base, EPT)])
    pltpu.sync_copy(denom_v, denom_out.at[wid])


def _s3_body(src_hbm, dst_hbm, ex_hbm, r_hbm, xs_hbm,
             outp_hbm,
             r_v, srcc_v, dstc_v, exc_v, coef_v, rows_v, zbuf_v, out_sh, sem):
    cid = lax.axis_index("c")
    sid = lax.axis_index("s")
    wid = sid * NC + cid
    base = wid * EPT
    pltpu.sync_copy(r_hbm, r_v)

    def zrow(i, c):
        for h in range(HID // L):
            zbuf_v[i, pl.ds(h * L, L)] = jnp.zeros((L,), jnp.float32)
        return c

    lax.fori_loop(0, ZR, zrow, 0)
    for k in range(RPT // ZR):
        pltpu.sync_copy(zbuf_v, out_sh.at[pl.ds(sid * RPT + k * ZR, ZR)])
    plsc.subcore_barrier()

    def chunk(c, carry):
        cb = base + c * CH
        pltpu.sync_copy(src_hbm.at[pl.ds(cb, CH)], srcc_v)
        pltpu.sync_copy(dst_hbm.at[pl.ds(cb, CH)], dstc_v)
        pltpu.sync_copy(ex_hbm.at[pl.ds(cb, CH)], exc_v)
        pltpu.async_copy(xs_hbm.at[srcc_v], rows_v, sem).wait()

        def cf(j, cc):
            sl = pl.ds(j * L, L)
            coef_v[sl] = exc_v[sl] * plsc.load_gather(r_v, [dstc_v[sl]])
            return cc

        lax.fori_loop(0, CH // L, cf, 0)

        def rw(w, cc):
            spl = plsc.load_gather(coef_v, [jnp.full((L,), w, jnp.int32)])
            for h in range(HID // L):
                sl2 = pl.ds(h * L, L)
                rows_v[w, sl2] = rows_v[w, sl2] * spl
            return cc

        lax.fori_loop(0, CH, rw, 0)
        pltpu.sync_copy(rows_v, out_sh.at[dstc_v], add=True)
        return carry

    lax.fori_loop(0, NCH, chunk, 0)
    plsc.subcore_barrier()
    pltpu.sync_copy(out_sh.at[pl.ds(sid * RPT, RPT)],
                    outp_hbm.at[cid, pl.ds(sid * RPT, RPT)])


def kernel(edge_attr, edge_index, entity_count, node_table, W, att_src,
           att_dst, We, att_edge, bias):
    f32 = jnp.float32
    valid = jnp.arange(N, dtype=jnp.int32) < entity_count
    x = jnp.where(valid[:, None], node_table[:N], 0.0).astype(f32)
    src = edge_index[0]
    dst = edge_index[1]
    att2 = jnp.stack([att_src, att_dst], axis=1)
    atte = att_edge[:, None]

    xs, a2, v2 = pl.pallas_call(
        _k1a_body,
        out_shape=[
            jax.ShapeDtypeStruct((N, HID), f32),
            jax.ShapeDtypeStruct((N, 2), f32),
            jax.ShapeDtypeStruct((HID, 1), f32),
        ],
    )(x, W, We, att2, atte)
    a_src = a2[:, 0]
    a_dst = a2[:, 1]

    EB = 12800
    ae2 = pl.pallas_call(
        _k1b_body,
        grid=(E // EB,),
        in_specs=[
            pl.BlockSpec((EB, HID), lambda i: (i, 0)),
            pl.BlockSpec((HID, 1), lambda i: (0, 0)),
        ],
        out_specs=pl.BlockSpec((EB, 1), lambda i: (i, 0)),
        out_shape=jax.ShapeDtypeStruct((E, 1), f32),
    )(edge_attr, v2)
    ae = ae2.reshape(E)

    mesh = plsc.VectorSubcoreMesh(core_axis_name="c", subcore_axis_name="s",
                                  num_cores=NC, num_subcores=NS)

    s1 = functools.partial(
        pl.kernel,
        out_type=[
            jax.ShapeDtypeStruct((E,), f32),
            jax.ShapeDtypeStruct((NW, N), f32),
        ],
        mesh=mesh,
        scratch_types=[
            pltpu.VMEM((N,), f32),
            pltpu.VMEM((N,), f32),
            pltpu.VMEM((EPT,), jnp.int32),
            pltpu.VMEM((EPT,), jnp.int32),
            pltpu.VMEM((EPT,), f32),
            pltpu.VMEM((EPT,), f32),
            pltpu.VMEM((N,), f32),
        ],
    )(_s1_body)
    alpha, amax_part = s1(a_src, a_dst, src, dst, ae)

    amax, alpha_s = pl.pallas_call(
        _k2_body,
        out_shape=[
            jax.ShapeDtypeStruct((N,), f32),
            jax.ShapeDtypeStruct((N,), f32),
        ],
    )(ae2, a_src, a_dst, amax_part)

    s2 = functools.partial(
        pl.kernel,
        out_type=[
            jax.ShapeDtypeStruct((E,), f32),
            jax.ShapeDtypeStruct((NW, N), f32),
        ],
        mesh=mesh,
        scratch_types=[
            pltpu.VMEM((N,), f32),
            pltpu.VMEM((EPT,), jnp.int32),
            pltpu.VMEM((EPT,), f32),
            pltpu.VMEM((EPT,), f32),
            pltpu.VMEM((N,), f32),
        ],
    )(_s2_body)
    ex, denom_part = s2(dst, alpha, amax)

    r, coef_s = pl.pallas_call(
        _k3_body,
        out_shape=[
            jax.ShapeDtypeStruct((N,), f32),
            jax.ShapeDtypeStruct((N,), f32),
        ],
    )(denom_part, amax, alpha_s)

    s3 = functools.partial(
        pl.kernel,
        out_type=[jax.ShapeDtypeStruct((NC, N, HID), f32)],
        mesh=mesh,
        scratch_types=[
            pltpu.VMEM((N,), f32),
            pltpu.VMEM((CH,), jnp.int32),
            pltpu.VMEM((CH,), jnp.int32),
            pltpu.VMEM((CH,), f32),
            pltpu.VMEM((CH,), f32),
            pltpu.VMEM((CH, HID), f32),
            pltpu.VMEM((ZR, HID), f32),
            pltpu.VMEM_SHARED((N, HID), f32),
            pltpu.SemaphoreType.DMA,
        ],
    )(_s3_body)
    out_part = s3(src, dst, ex, r, xs)
    if isinstance(out_part, (tuple, list)):
        out_part = out_part[0]

    out = pl.pallas_call(
        _k4_body,
        out_shape=jax.ShapeDtypeStruct((N, HID), f32),
    )(out_part, xs, coef_s, bias)
    return out


# speedup vs baseline: 13.0860x; 13.0860x over previous
import functools

import jax
import jax.numpy as jnp
from jax import lax
from jax.experimental import pallas as pl
from jax.experimental.pallas import tpu as pltpu
from jax.experimental.pallas import tpu_sc as plsc

HID = 128
N = 10000
E = 320000
NC, NS, L = 2, 16, 16
NW = NC * NS
EPT = E // NW
CH = 80
NCH = EPT // CH
RPT = N // NS
ZR = 125
NEG = -1e30


def _k1a_body(x_ref, w_ref, we_ref, att2_ref, atte_ref, xs_ref, a2_ref, v_ref):
    xs = jnp.dot(x_ref[...], w_ref[...], preferred_element_type=jnp.float32)
    xs_ref[...] = xs
    a2_ref[...] = jnp.dot(xs, att2_ref[...], preferred_element_type=jnp.float32)
    v_ref[...] = jnp.dot(we_ref[...], atte_ref[...],
                         preferred_element_type=jnp.float32)


def _k1b_body(ea_ref, v_ref, ae_ref):
    ae_ref[...] = jnp.dot(ea_ref[...], v_ref[...],
                          preferred_element_type=jnp.float32)


def _k2_body(ae_ref, asrc_ref, adst_ref, part_ref, amax_ref, als_ref):
    mean = jnp.sum(ae_ref[...]) * (1.0 / E)
    a = asrc_ref[...] + adst_ref[...] + mean
    als = jnp.where(a >= 0, a, 0.2 * a)
    als_ref[...] = als
    amax_ref[...] = jnp.maximum(jnp.max(part_ref[...], axis=0), als)


def _k3_body(part_ref, amax_ref, als_ref, r_ref, cs_ref):
    ex_s = jnp.exp(als_ref[...] - amax_ref[...])
    denom = jnp.sum(part_ref[...], axis=0) + ex_s
    r = 1.0 / (denom + 1e-16)
    r_ref[...] = r
    cs_ref[...] = ex_s * r


def _k4_body(part_ref, xs_ref, cs_ref, bias_ref, out_ref):
    acc = part_ref[0] + part_ref[1]
    out_ref[...] = acc + cs_ref[...][:, None] * xs_ref[...] + bias_ref[...]


def _leaky(x):
    return jnp.where(x >= 0, x, 0.2 * x)


def _s1_body(asrc_hbm, adst_hbm, src_hbm, dst_hbm, ae_hbm,
             alpha_out, amax_out,
             asrc_v, adst_v, src_v, dst_v, ae_v, alpha_v, amax_v):
    cid = lax.axis_index("c")
    sid = lax.axis_index("s")
    wid = sid * NC + cid
    base = wid * EPT
    pltpu.sync_copy(asrc_hbm, asrc_v)
    pltpu.sync_copy(adst_hbm, adst_v)
    pltpu.sync_copy(src_hbm.at[pl.ds(base, EPT)], src_v)
    pltpu.sync_copy(dst_hbm.at[pl.ds(base, EPT)], dst_v)
    pltpu.sync_copy(ae_hbm.at[pl.ds(base, EPT)], ae_v)

    def initb(i, c):
        amax_v[pl.ds(i * L, L)] = jnp.full((L,), NEG, jnp.float32)
        return c

    lax.fori_loop(0, N // L, initb, 0)

    def edge_block(i, c):
        sl = pl.ds(i * L, L)
        s = src_v[sl]
        d = dst_v[sl]
        a = (plsc.load_gather(asrc_v, [s]) + plsc.load_gather(adst_v, [d])
             + ae_v[sl])
        al = _leaky(a)
        alpha_v[sl] = al

        def cond(cur):
            return jnp.any(al > cur)

        def body(cur):
            plsc.store_scatter(amax_v, [d], al, mask=al > cur)
            return plsc.load_gather(amax_v, [d])

        lax.while_loop(cond, body, plsc.load_gather(amax_v, [d]))
        return c

    lax.fori_loop(0, EPT // L, edge_block, 0)
    pltpu.sync_copy(alpha_v, alpha_out.at[pl.ds(base, EPT)])
    pltpu.sync_copy(amax_v, amax_out.at[wid])


def _s2_body(dst_hbm, alpha_hbm, amax_hbm,
             ex_out, denom_out,
             amax_v, dst_v, alpha_v, ex_v, denom_v):
    cid = lax.axis_index("c")
    sid = lax.axis_index("s")
    wid = sid * NC + cid
    base = wid * EPT
    pltpu.sync_copy(amax_hbm, amax_v)
    pltpu.sync_copy(dst_hbm.at[pl.ds(base, EPT)], dst_v)
    pltpu.sync_copy(alpha_hbm.at[pl.ds(base, EPT)], alpha_v)

    def initb(i, c):
        denom_v[pl.ds(i * L, L)] = jnp.zeros((L,), jnp.float32)
        return c

    lax.fori_loop(0, N // L, initb, 0)

    def edge_block(i, c):
        sl = pl.ds(i * L, L)
        d = dst_v[sl]
        am = plsc.load_gather(amax_v, [d])
        e = jnp.exp(alpha_v[sl] - am)
        ex_v[sl] = e
        plsc.addupdate_scatter(denom_v, [d], e)
        return c

    lax.fori_loop(0, EPT // L, edge_block, 0)
    pltpu.sync_copy(ex_v, ex_out.at[pl.ds(base, EPT)])
    pltpu.sync_copy(denom_v, denom_out.at[wid])


def _s3_body(src_hbm, dst_hbm, ex_hbm, r_hbm, xs_hbm,
             outp_hbm,
             r_v, srcc_v, dstc_v, exc_v, coef_v, rows_v, zbuf_v, out_sh, sem):
    cid = lax.axis_index("c")
    sid = lax.axis_index("s")
    wid = sid * NC + cid
    base = wid * EPT
    pltpu.sync_copy(r_hbm, r_v)

    def zrow(i, c):
        for h in range(HID // L):
            zbuf_v[i, pl.ds(h * L, L)] = jnp.zeros((L,), jnp.float32)
        return c

    lax.fori_loop(0, ZR, zrow, 0)
    for k in range(RPT // ZR):
        pltpu.sync_copy(zbuf_v, out_sh.at[pl.ds(sid * RPT + k * ZR, ZR)])
    plsc.subcore_barrier()

    def chunk(c, carry):
        cb = base + c * CH
        pltpu.sync_copy(src_hbm.at[pl.ds(cb, CH)], srcc_v)
        pltpu.sync_copy(dst_hbm.at[pl.ds(cb, CH)], dstc_v)
        pltpu.sync_copy(ex_hbm.at[pl.ds(cb, CH)], exc_v)
        pltpu.async_copy(xs_hbm.at[srcc_v], rows_v, sem).wait()

        def cf(j, cc):
            sl = pl.ds(j * L, L)
            coef_v[sl] = exc_v[sl] * plsc.load_gather(r_v, [dstc_v[sl]])
            return cc

        lax.fori_loop(0, CH // L, cf, 0)

        def rw(w, cc):
            spl = plsc.load_gather(coef_v, [jnp.full((L,), w, jnp.int32)])
            for h in range(HID // L):
                sl2 = pl.ds(h * L, L)
                rows_v[w, sl2] = rows_v[w, sl2] * spl
            return cc

        lax.fori_loop(0, CH, rw, 0)
        pltpu.sync_copy(rows_v, out_sh.at[dstc_v], add=True)
        return carry

    lax.fori_loop(0, NCH, chunk, 0)
    plsc.subcore_barrier()
    pltpu.sync_copy(out_sh.at[pl.ds(sid * RPT, RPT)],
                    outp_hbm.at[cid, pl.ds(sid * RPT, RPT)])


def kernel(edge_attr, edge_index, entity_count, node_table, W, att_src,
           att_dst, We, att_edge, bias):
    f32 = jnp.float32
    valid = jnp.arange(N, dtype=jnp.int32) < entity_count
    x = jnp.where(valid[:, None], node_table[:N], 0.0).astype(f32)
    src = edge_index[0]
    dst = edge_index[1]
    att2 = jnp.stack([att_src, att_dst], axis=1)
    atte = att_edge[:, None]

    xs, a2, v2 = pl.pallas_call(
        _k1a_body,
        out_shape=[
            jax.ShapeDtypeStruct((N, HID), f32),
            jax.ShapeDtypeStruct((N, 2), f32),
            jax.ShapeDtypeStruct((HID, 1), f32),
        ],
    )(x, W, We, att2, atte)
    a_src = a2[:, 0]
    a_dst = a2[:, 1]

    EB = 12800
    ae2 = pl.pallas_call(
        _k1b_body,
        grid=(E // EB,),
        in_specs=[
            pl.BlockSpec((EB, HID), lambda i: (i, 0)),
            pl.BlockSpec((HID, 1), lambda i: (0, 0)),
        ],
        out_specs=pl.BlockSpec((EB, 1), lambda i: (i, 0)),
        out_shape=jax.ShapeDtypeStruct((E, 1), f32),
    )(edge_attr, v2)
    ae = ae2.reshape(E)

    mesh = plsc.VectorSubcoreMesh(core_axis_name="c", subcore_axis_name="s",
                                  num_cores=NC, num_subcores=NS)

    s1 = functools.partial(
        pl.kernel,
        out_type=[
            jax.ShapeDtypeStruct((E,), f32),
            jax.ShapeDtypeStruct((NW, N), f32),
        ],
        mesh=mesh,
        compiler_params=pltpu.CompilerParams(needs_layout_passes=False),
        scratch_types=[
            pltpu.VMEM((N,), f32),
            pltpu.VMEM((N,), f32),
            pltpu.VMEM((EPT,), jnp.int32),
            pltpu.VMEM((EPT,), jnp.int32),
            pltpu.VMEM((EPT,), f32),
            pltpu.VMEM((EPT,), f32),
            pltpu.VMEM((N,), f32),
        ],
    )(_s1_body)
    alpha, amax_part = s1(a_src, a_dst, src, dst, ae)

    amax, alpha_s = pl.pallas_call(
        _k2_body,
        out_shape=[
            jax.ShapeDtypeStruct((N,), f32),
            jax.ShapeDtypeStruct((N,), f32),
        ],
    )(ae2.reshape(E // HID, HID), a_src, a_dst, amax_part)

    s2 = functools.partial(
        pl.kernel,
        out_type=[
            jax.ShapeDtypeStruct((E,), f32),
            jax.ShapeDtypeStruct((NW, N), f32),
        ],
        mesh=mesh,
        compiler_params=pltpu.CompilerParams(needs_layout_passes=False),
        scratch_types=[
            pltpu.VMEM((N,), f32),
            pltpu.VMEM((EPT,), jnp.int32),
            pltpu.VMEM((EPT,), f32),
            pltpu.VMEM((EPT,), f32),
            pltpu.VMEM((N,), f32),
        ],
    )(_s2_body)
    ex, denom_part = s2(dst, alpha, amax)

    r, coef_s = pl.pallas_call(
        _k3_body,
        out_shape=[
            jax.ShapeDtypeStruct((N,), f32),
            jax.ShapeDtypeStruct((N,), f32),
        ],
    )(denom_part, amax, alpha_s)

    s3 = functools.partial(
        pl.kernel,
        out_type=[jax.ShapeDtypeStruct((NC, N, HID), f32)],
        mesh=mesh,
        compiler_params=pltpu.CompilerParams(needs_layout_passes=False,
                                             use_tc_tiling_on_sc=False),
        scratch_types=[
            pltpu.VMEM((N,), f32),
            pltpu.VMEM((CH,), jnp.int32),
            pltpu.VMEM((CH,), jnp.int32),
            pltpu.VMEM((CH,), f32),
            pltpu.VMEM((CH,), f32),
            pltpu.VMEM((CH, HID), f32),
            pltpu.VMEM((ZR, HID), f32),
            pltpu.VMEM_SHARED((N, HID), f32),
            pltpu.SemaphoreType.DMA,
        ],
    )(_s3_body)
    out_part = s3(src, dst, ex, r, xs)
    if isinstance(out_part, (tuple, list)):
        out_part = out_part[0]

    out = pl.pallas_call(
        _k4_body,
        out_shape=jax.ShapeDtypeStruct((N, HID), f32),
    )(out_part, xs, coef_s, bias)
    return out


# speedup vs baseline: 16.4361x; 1.2560x over previous
import functools

import jax
import jax.numpy as jnp
from jax import lax
from jax.experimental import pallas as pl
from jax.experimental.pallas import tpu as pltpu
from jax.experimental.pallas import tpu_sc as plsc

HID = 128
N = 10000
E = 320000
NC, NS, L = 2, 16, 16
NW = NC * NS
EPT = E // NW
CH = 80
NCH = EPT // CH
WB = 624
NEG = -1e30


def _k1a_body(x_ref, w_ref, we_ref, att2_ref, atte_ref, xs_ref, a2_ref, v_ref):
    xs = jnp.dot(x_ref[...], w_ref[...], preferred_element_type=jnp.float32)
    xs_ref[...] = xs
    a2_ref[...] = jnp.dot(xs, att2_ref[...], preferred_element_type=jnp.float32)
    v_ref[...] = jnp.dot(we_ref[...], atte_ref[...],
                         preferred_element_type=jnp.float32)


def _k1b_body(ea_ref, v_ref, ae_ref):
    ae_ref[...] = jnp.dot(ea_ref[...], v_ref[...],
                          preferred_element_type=jnp.float32)


def _k2_body(ae_ref, asrc_ref, adst_ref, part_ref, amax_ref, als_ref):
    mean = jnp.sum(ae_ref[...]) * (1.0 / E)
    a = asrc_ref[...] + adst_ref[...] + mean
    als = jnp.where(a >= 0, a, 0.2 * a)
    als_ref[...] = als
    amax_ref[...] = jnp.maximum(jnp.max(part_ref[...], axis=0), als)


def _k3_body(part_ref, amax_ref, als_ref, r_ref, cs_ref):
    ex_s = jnp.exp(als_ref[...] - amax_ref[...])
    denom = jnp.sum(part_ref[...], axis=0) + ex_s
    r = 1.0 / (denom + 1e-16)
    r_ref[...] = r
    cs_ref[...] = ex_s * r


def _k4_body(part_ref, xs_ref, cs_ref, bias_ref, out_ref):
    acc = part_ref[0] + part_ref[1]
    out_ref[...] = acc + cs_ref[...][:, None] * xs_ref[...] + bias_ref[...]


def _leaky(x):
    return jnp.where(x >= 0, x, 0.2 * x)


def _s1_body(asrc_hbm, adst_hbm, src_hbm, dst_hbm, ae_hbm,
             alpha_out, amax_out,
             asrc_v, adst_v, src_v, dst_v, ae_v, alpha_v, amax_v):
    cid = lax.axis_index("c")
    sid = lax.axis_index("s")
    wid = sid * NC + cid
    base = wid * EPT
    pltpu.sync_copy(asrc_hbm, asrc_v)
    pltpu.sync_copy(adst_hbm, adst_v)
    pltpu.sync_copy(src_hbm.at[pl.ds(base, EPT)], src_v)
    pltpu.sync_copy(dst_hbm.at[pl.ds(base, EPT)], dst_v)
    pltpu.sync_copy(ae_hbm.at[pl.ds(base, EPT)], ae_v)

    def initb(i, c):
        amax_v[pl.ds(i * L, L)] = jnp.full((L,), NEG, jnp.float32)
        return c

    lax.fori_loop(0, N // L, initb, 0)

    def edge_block(i, c):
        sl = pl.ds(i * L, L)
        s = src_v[sl]
        d = dst_v[sl]
        a = (plsc.load_gather(asrc_v, [s]) + plsc.load_gather(adst_v, [d])
             + ae_v[sl])
        al = _leaky(a)
        alpha_v[sl] = al

        def cond(cur):
            return jnp.any(al > cur)

        def body(cur):
            plsc.store_scatter(amax_v, [d], al, mask=al > cur)
            return plsc.load_gather(amax_v, [d])

        lax.while_loop(cond, body, plsc.load_gather(amax_v, [d]))
        return c

    lax.fori_loop(0, EPT // L, edge_block, 0)
    pltpu.sync_copy(alpha_v, alpha_out.at[pl.ds(base, EPT)])
    pltpu.sync_copy(amax_v, amax_out.at[wid])


def _s2_body(dst_hbm, alpha_hbm, amax_hbm,
             ex_out, denom_out,
             amax_v, dst_v, alpha_v, ex_v, denom_v):
    cid = lax.axis_index("c")
    sid = lax.axis_index("s")
    wid = sid * NC + cid
    base = wid * EPT
    pltpu.sync_copy(amax_hbm, amax_v)
    pltpu.sync_copy(dst_hbm.at[pl.ds(base, EPT)], dst_v)
    pltpu.sync_copy(alpha_hbm.at[pl.ds(base, EPT)], alpha_v)

    def initb(i, c):
        denom_v[pl.ds(i * L, L)] = jnp.zeros((L,), jnp.float32)
        return c

    lax.fori_loop(0, N // L, initb, 0)

    def edge_block(i, c):
        sl = pl.ds(i * L, L)
        d = dst_v[sl]
        am = plsc.load_gather(amax_v, [d])
        e = jnp.exp(alpha_v[sl] - am)
        ex_v[sl] = e
        plsc.addupdate_scatter(denom_v, [d], e)
        return c

    lax.fori_loop(0, EPT // L, edge_block, 0)
    pltpu.sync_copy(ex_v, ex_out.at[pl.ds(base, EPT)])
    pltpu.sync_copy(denom_v, denom_out.at[wid])


def _s3_body(edata_hbm, r_hbm, xs_hbm,
             outp_hbm,
             r_v, ech0_v, ech1_v, coefc0_v, coefc1_v, rows0_v, rows1_v,
             out_sh, gsem0, gsem1, ssem0, ssem1):
    cid = lax.axis_index("c")
    sid = lax.axis_index("s")
    wid = sid * NC + cid
    pltpu.sync_copy(r_hbm, r_v)

    def zrow(i, c):
        for h in range(HID // L):
            rows0_v[i, pl.ds(h * L, L)] = jnp.zeros((L,), jnp.float32)
        return c

    lax.fori_loop(0, CH, zrow, 0)
    for k in range(WB // CH):
        pltpu.sync_copy(rows0_v, out_sh.at[pl.ds(sid * WB + k * CH, CH)])
    pltpu.sync_copy(rows0_v.at[pl.ds(0, WB - (WB // CH) * CH)],
                    out_sh.at[pl.ds(sid * WB + (WB // CH) * CH,
                                    WB - (WB // CH) * CH)])

    @pl.when(sid == 0)
    def _():
        pltpu.sync_copy(rows0_v.at[pl.ds(0, N - NS * WB)],
                        out_sh.at[pl.ds(NS * WB, N - NS * WB)])

    plsc.subcore_barrier()

    def coefs(ech, coefc):
        for j in range(CH // L):
            sl = pl.ds(j * L, L)
            e = plsc.bitcast(ech[2, sl], jnp.float32)
            coefc[sl] = e * plsc.load_gather(r_v, [ech[1, sl]])

    def scale(buf, coefc):
        def grp(j, cc):
            for w in range(L):
                idxr = jnp.full((L,), j * L + w, jnp.int32)
                spl = plsc.load_gather(coefc, [idxr])
                row = j * L + w
                for h in range(HID // L):
                    sl2 = pl.ds(h * L, L)
                    buf[row, sl2] = buf[row, sl2] * spl
            return cc

        lax.fori_loop(0, CH // L, grp, 0)

    def pair(i, carry):
        c0 = 2 * i
        c1 = c0 + 1

        @pl.when(i > 0)
        def _():
            pltpu.make_async_copy(rows0_v, out_sh.at[ech0_v.at[1]],
                                  ssem0).wait()
            pltpu.make_async_copy(rows1_v, out_sh.at[ech1_v.at[1]],
                                  ssem1).wait()

        pltpu.sync_copy(edata_hbm.at[wid, c0], ech0_v)
        pltpu.sync_copy(edata_hbm.at[wid, c1], ech1_v)
        g0 = pltpu.async_copy(xs_hbm.at[ech0_v.at[0]], rows0_v, gsem0)
        g1 = pltpu.async_copy(xs_hbm.at[ech1_v.at[0]], rows1_v, gsem1)
        coefs(ech0_v, coefc0_v)
        coefs(ech1_v, coefc1_v)
        g0.wait()
        scale(rows0_v, coefc0_v)
        pltpu.async_copy(rows0_v, out_sh.at[ech0_v.at[1]], ssem0, add=True)
        g1.wait()
        scale(rows1_v, coefc1_v)
        pltpu.async_copy(rows1_v, out_sh.at[ech1_v.at[1]], ssem1, add=True)
        return carry

    lax.fori_loop(0, NCH // 2, pair, 0)
    pltpu.make_async_copy(rows0_v, out_sh.at[ech0_v.at[1]], ssem0).wait()
    pltpu.make_async_copy(rows1_v, out_sh.at[ech1_v.at[1]], ssem1).wait()
    pltpu.sync_copy(edata_hbm.at[wid, NCH - 1], ech0_v)
    g0 = pltpu.async_copy(xs_hbm.at[ech0_v.at[0]], rows0_v, gsem0)
    coefs(ech0_v, coefc0_v)
    g0.wait()
    scale(rows0_v, coefc0_v)
    pltpu.sync_copy(rows0_v, out_sh.at[ech0_v.at[1]], add=True)

    plsc.subcore_barrier()
    pltpu.sync_copy(out_sh.at[pl.ds(sid * WB, WB)],
                    outp_hbm.at[cid, pl.ds(sid * WB, WB)])

    @pl.when(sid == 0)
    def _():
        pltpu.sync_copy(out_sh.at[pl.ds(NS * WB, N - NS * WB)],
                        outp_hbm.at[cid, pl.ds(NS * WB, N - NS * WB)])


def kernel(edge_attr, edge_index, entity_count, node_table, W, att_src,
           att_dst, We, att_edge, bias):
    f32 = jnp.float32
    valid = jnp.arange(N, dtype=jnp.int32) < entity_count
    x = jnp.where(valid[:, None], node_table[:N], 0.0).astype(f32)
    src = edge_index[0]
    dst = edge_index[1]
    att2 = jnp.stack([att_src, att_dst], axis=1)
    atte = att_edge[:, None]

    xs, a2, v2 = pl.pallas_call(
        _k1a_body,
        out_shape=[
            jax.ShapeDtypeStruct((N, HID), f32),
            jax.ShapeDtypeStruct((N, 2), f32),
            jax.ShapeDtypeStruct((HID, 1), f32),
        ],
    )(x, W, We, att2, atte)
    a_src = a2[:, 0]
    a_dst = a2[:, 1]

    EB = 12800
    ae2 = pl.pallas_call(
        _k1b_body,
        grid=(E // EB,),
        in_specs=[
            pl.BlockSpec((EB, HID), lambda i: (i, 0)),
            pl.BlockSpec((HID, 1), lambda i: (0, 0)),
        ],
        out_specs=pl.BlockSpec((EB, 1), lambda i: (i, 0)),
        out_shape=jax.ShapeDtypeStruct((E, 1), f32),
    )(edge_attr, v2)
    ae = ae2.reshape(E)

    mesh = plsc.VectorSubcoreMesh(core_axis_name="c", subcore_axis_name="s",
                                  num_cores=NC, num_subcores=NS)

    s1 = functools.partial(
        pl.kernel,
        out_type=[
            jax.ShapeDtypeStruct((E,), f32),
            jax.ShapeDtypeStruct((NW, N), f32),
        ],
        mesh=mesh,
        compiler_params=pltpu.CompilerParams(needs_layout_passes=False),
        scratch_types=[
            pltpu.VMEM((N,), f32),
            pltpu.VMEM((N,), f32),
            pltpu.VMEM((EPT,), jnp.int32),
            pltpu.VMEM((EPT,), jnp.int32),
            pltpu.VMEM((EPT,), f32),
            pltpu.VMEM((EPT,), f32),
            pltpu.VMEM((N,), f32),
        ],
    )(_s1_body)
    alpha, amax_part = s1(a_src, a_dst, src, dst, ae)

    amax, alpha_s = pl.pallas_call(
        _k2_body,
        out_shape=[
            jax.ShapeDtypeStruct((N,), f32),
            jax.ShapeDtypeStruct((N,), f32),
        ],
    )(ae2.reshape(E // HID, HID), a_src, a_dst, amax_part)

    s2 = functools.partial(
        pl.kernel,
        out_type=[
            jax.ShapeDtypeStruct((E,), f32),
            jax.ShapeDtypeStruct((NW, N), f32),
        ],
        mesh=mesh,
        compiler_params=pltpu.CompilerParams(needs_layout_passes=False),
        scratch_types=[
            pltpu.VMEM((N,), f32),
            pltpu.VMEM((EPT,), jnp.int32),
            pltpu.VMEM((EPT,), f32),
            pltpu.VMEM((EPT,), f32),
            pltpu.VMEM((N,), f32),
        ],
    )(_s2_body)
    ex, denom_part = s2(dst, alpha, amax)

    r, coef_s = pl.pallas_call(
        _k3_body,
        out_shape=[
            jax.ShapeDtypeStruct((N,), f32),
            jax.ShapeDtypeStruct((N,), f32),
        ],
    )(denom_part, amax, alpha_s)

    s3 = functools.partial(
        pl.kernel,
        out_type=[jax.ShapeDtypeStruct((NC, N, HID), f32)],
        mesh=mesh,
        compiler_params=pltpu.CompilerParams(needs_layout_passes=False),
        scratch_types=[
            pltpu.VMEM((N,), f32),
            pltpu.VMEM((3, CH), jnp.int32),
            pltpu.VMEM((3, CH), jnp.int32),
            pltpu.VMEM((CH,), f32),
            pltpu.VMEM((CH,), f32),
            pltpu.VMEM((CH, HID), f32),
            pltpu.VMEM((CH, HID), f32),
            pltpu.VMEM_SHARED((N, HID), f32),
            pltpu.SemaphoreType.DMA,
            pltpu.SemaphoreType.DMA,
            pltpu.SemaphoreType.DMA,
            pltpu.SemaphoreType.DMA,
        ],
    )(_s3_body)
    ex_bits = lax.bitcast_convert_type(ex, jnp.int32)
    edata = jnp.stack([src.reshape(NW, NCH, CH), dst.reshape(NW, NCH, CH),
                       ex_bits.reshape(NW, NCH, CH)], axis=2)
    out_part = s3(edata, r, xs)
    if isinstance(out_part, (tuple, list)):
        out_part = out_part[0]

    out = pl.pallas_call(
        _k4_body,
        out_shape=jax.ShapeDtypeStruct((N, HID), f32),
    )(out_part, xs, coef_s, bias)
    return out


# speedup vs baseline: 17.4303x; 1.0605x over previous
import functools

import jax
import jax.numpy as jnp
from jax import lax
from jax.experimental import pallas as pl
from jax.experimental.pallas import tpu as pltpu
from jax.experimental.pallas import tpu_sc as plsc

HID = 128
N = 10000
E = 320000
NC, NS, L = 2, 16, 16
NW = NC * NS
EPT = E // NW
CH = 80
NCH = EPT // CH
WB = 624
NEG = -1e30


def _k1a_body(x_ref, w_ref, we_ref, att2_ref, atte_ref, xs_ref, a2_ref, v_ref):
    xs = jnp.dot(x_ref[...], w_ref[...], preferred_element_type=jnp.float32)
    xs_ref[...] = xs
    a2_ref[...] = jnp.dot(xs, att2_ref[...], preferred_element_type=jnp.float32)
    v_ref[...] = jnp.dot(we_ref[...], atte_ref[...],
                         preferred_element_type=jnp.float32)


def _k1b_body(ea_ref, v_ref, a2_ref, ae_ref, bound_ref, als_ref, sm_ref):
    i = pl.program_id(0)
    aeb = jnp.dot(ea_ref[...], v_ref[...], preferred_element_type=jnp.float32)
    ae_ref[...] = aeb
    bm = jnp.max(aeb)
    bs = jnp.sum(aeb)

    @pl.when(i == 0)
    def _():
        sm_ref[0] = bm
        sm_ref[1] = bs

    @pl.when(i > 0)
    def _():
        sm_ref[0] = jnp.maximum(sm_ref[0], bm)
        sm_ref[1] = sm_ref[1] + bs

    @pl.when(i == pl.num_programs(0) - 1)
    def _():
        a_src = a2_ref[...][:, 0]
        a_dst = a2_ref[...][:, 1]
        mean = sm_ref[1] * (1.0 / E)
        pre = a_dst + (jnp.max(a_src) + sm_ref[0])
        bound_ref[...] = jnp.where(pre >= 0, pre, 0.2 * pre)
        a = a_src + a_dst + mean
        als_ref[...] = jnp.where(a >= 0, a, 0.2 * a)


def _k3_body(part_ref, bound_ref, als_ref, r_ref, cs_ref):
    ex_s = jnp.exp(als_ref[...] - bound_ref[...])
    denom = jnp.sum(part_ref[...], axis=0) + ex_s
    r = 1.0 / (denom + 1e-16)
    r_ref[...] = r
    cs_ref[...] = ex_s * r


def _k4_body(part_ref, xs_ref, cs_ref, bias_ref, out_ref):
    acc = part_ref[0] + part_ref[1]
    out_ref[...] = acc + cs_ref[...][:, None] * xs_ref[...] + bias_ref[...]


def _leaky(x):
    return jnp.where(x >= 0, x, 0.2 * x)


def _s12_body(asrc_hbm, adst_hbm, src_hbm, dst_hbm, ae_hbm, bound_hbm,
              ex_out, denom_out,
              asrc_v, adst_v, bound_v, src_v, dst_v, ae_v, ex_v, denom_v):
    cid = lax.axis_index("c")
    sid = lax.axis_index("s")
    wid = sid * NC + cid
    base = wid * EPT
    pltpu.sync_copy(asrc_hbm, asrc_v)
    pltpu.sync_copy(adst_hbm, adst_v)
    pltpu.sync_copy(bound_hbm, bound_v)
    pltpu.sync_copy(src_hbm.at[pl.ds(base, EPT)], src_v)
    pltpu.sync_copy(dst_hbm.at[pl.ds(base, EPT)], dst_v)
    pltpu.sync_copy(ae_hbm.at[pl.ds(base, EPT)], ae_v)

    def initb(i, c):
        denom_v[pl.ds(i * L, L)] = jnp.zeros((L,), jnp.float32)
        return c

    lax.fori_loop(0, N // L, initb, 0)

    def edge_block(i, c):
        sl = pl.ds(i * L, L)
        s = src_v[sl]
        d = dst_v[sl]
        a = (plsc.load_gather(asrc_v, [s]) + plsc.load_gather(adst_v, [d])
             + ae_v[sl])
        al = _leaky(a)
        e = jnp.exp(al - plsc.load_gather(bound_v, [d]))
        ex_v[sl] = e
        plsc.addupdate_scatter(denom_v, [d], e)
        return c

    lax.fori_loop(0, EPT // L, edge_block, 0)
    pltpu.sync_copy(ex_v, ex_out.at[pl.ds(base, EPT)])
    pltpu.sync_copy(denom_v, denom_out.at[wid])


def _s3_body(edata_hbm, r_hbm, xs_hbm,
             outp_hbm,
             r_v, ech0_v, ech1_v, coefc0_v, coefc1_v, rows0_v, rows1_v,
             out_sh, gsem0, gsem1, ssem0, ssem1):
    cid = lax.axis_index("c")
    sid = lax.axis_index("s")
    wid = sid * NC + cid
    pltpu.sync_copy(r_hbm, r_v)

    def zrow(i, c):
        for h in range(HID // L):
            rows0_v[i, pl.ds(h * L, L)] = jnp.zeros((L,), jnp.float32)
        return c

    lax.fori_loop(0, CH, zrow, 0)
    for k in range(WB // CH):
        pltpu.sync_copy(rows0_v, out_sh.at[pl.ds(sid * WB + k * CH, CH)])
    pltpu.sync_copy(rows0_v.at[pl.ds(0, WB - (WB // CH) * CH)],
                    out_sh.at[pl.ds(sid * WB + (WB // CH) * CH,
                                    WB - (WB // CH) * CH)])

    @pl.when(sid == 0)
    def _():
        pltpu.sync_copy(rows0_v.at[pl.ds(0, N - NS * WB)],
                        out_sh.at[pl.ds(NS * WB, N - NS * WB)])

    plsc.subcore_barrier()

    def coefs(ech, coefc):
        for j in range(CH // L):
            sl = pl.ds(j * L, L)
            e = plsc.bitcast(ech[2, sl], jnp.float32)
            coefc[sl] = e * plsc.load_gather(r_v, [ech[1, sl]])

    def scale(buf, coefc):
        def grp(j, cc):
            for w in range(L):
                idxr = jnp.full((L,), j * L + w, jnp.int32)
                spl = plsc.load_gather(coefc, [idxr])
                row = j * L + w
                for h in range(HID // L):
                    sl2 = pl.ds(h * L, L)
                    buf[row, sl2] = buf[row, sl2] * spl
            return cc

        lax.fori_loop(0, CH // L, grp, 0)

    def pair(i, carry):
        c0 = 2 * i
        c1 = c0 + 1

        @pl.when(i > 0)
        def _():
            pltpu.make_async_copy(rows0_v, out_sh.at[ech0_v.at[1]],
                                  ssem0).wait()
            pltpu.make_async_copy(rows1_v, out_sh.at[ech1_v.at[1]],
                                  ssem1).wait()

        pltpu.sync_copy(edata_hbm.at[wid, c0], ech0_v)
        pltpu.sync_copy(edata_hbm.at[wid, c1], ech1_v)
        g0 = pltpu.async_copy(xs_hbm.at[ech0_v.at[0]], rows0_v, gsem0)
        g1 = pltpu.async_copy(xs_hbm.at[ech1_v.at[0]], rows1_v, gsem1)
        coefs(ech0_v, coefc0_v)
        coefs(ech1_v, coefc1_v)
        g0.wait()
        scale(rows0_v, coefc0_v)
        pltpu.async_copy(rows0_v, out_sh.at[ech0_v.at[1]], ssem0, add=True)
        g1.wait()
        scale(rows1_v, coefc1_v)
        pltpu.async_copy(rows1_v, out_sh.at[ech1_v.at[1]], ssem1, add=True)
        return carry

    lax.fori_loop(0, NCH // 2, pair, 0)
    pltpu.make_async_copy(rows0_v, out_sh.at[ech0_v.at[1]], ssem0).wait()
    pltpu.make_async_copy(rows1_v, out_sh.at[ech1_v.at[1]], ssem1).wait()
    pltpu.sync_copy(edata_hbm.at[wid, NCH - 1], ech0_v)
    g0 = pltpu.async_copy(xs_hbm.at[ech0_v.at[0]], rows0_v, gsem0)
    coefs(ech0_v, coefc0_v)
    g0.wait()
    scale(rows0_v, coefc0_v)
    pltpu.sync_copy(rows0_v, out_sh.at[ech0_v.at[1]], add=True)

    plsc.subcore_barrier()
    pltpu.sync_copy(out_sh.at[pl.ds(sid * WB, WB)],
                    outp_hbm.at[cid, pl.ds(sid * WB, WB)])

    @pl.when(sid == 0)
    def _():
        pltpu.sync_copy(out_sh.at[pl.ds(NS * WB, N - NS * WB)],
                        outp_hbm.at[cid, pl.ds(NS * WB, N - NS * WB)])


def kernel(edge_attr, edge_index, entity_count, node_table, W, att_src,
           att_dst, We, att_edge, bias):
    f32 = jnp.float32
    valid = jnp.arange(N, dtype=jnp.int32) < entity_count
    x = jnp.where(valid[:, None], node_table[:N], 0.0).astype(f32)
    src = edge_index[0]
    dst = edge_index[1]
    att2 = jnp.stack([att_src, att_dst], axis=1)
    atte = att_edge[:, None]

    xs, a2, v2 = pl.pallas_call(
        _k1a_body,
        out_shape=[
            jax.ShapeDtypeStruct((N, HID), f32),
            jax.ShapeDtypeStruct((N, 2), f32),
            jax.ShapeDtypeStruct((HID, 1), f32),
        ],
    )(x, W, We, att2, atte)
    a_src = a2[:, 0]
    a_dst = a2[:, 1]

    EB = 12800
    ae2, bound, alpha_s = pl.pallas_call(
        _k1b_body,
        grid=(E // EB,),
        in_specs=[
            pl.BlockSpec((EB, HID), lambda i: (i, 0)),
            pl.BlockSpec((HID, 1), lambda i: (0, 0)),
            pl.BlockSpec((N, 2), lambda i: (0, 0)),
        ],
        out_specs=[
            pl.BlockSpec((EB, 1), lambda i: (i, 0)),
            pl.BlockSpec((N,), lambda i: (0,)),
            pl.BlockSpec((N,), lambda i: (0,)),
        ],
        out_shape=[
            jax.ShapeDtypeStruct((E, 1), f32),
            jax.ShapeDtypeStruct((N,), f32),
            jax.ShapeDtypeStruct((N,), f32),
        ],
        scratch_shapes=[pltpu.SMEM((2,), f32)],
    )(edge_attr, v2, a2)
    ae = ae2.reshape(E)

    mesh = plsc.VectorSubcoreMesh(core_axis_name="c", subcore_axis_name="s",
                                  num_cores=NC, num_subcores=NS)

    s12 = functools.partial(
        pl.kernel,
        out_type=[
            jax.ShapeDtypeStruct((E,), f32),
            jax.ShapeDtypeStruct((NW, N), f32),
        ],
        mesh=mesh,
        compiler_params=pltpu.CompilerParams(needs_layout_passes=False),
        scratch_types=[
            pltpu.VMEM((N,), f32),
            pltpu.VMEM((N,), f32),
            pltpu.VMEM((N,), f32),
            pltpu.VMEM((EPT,), jnp.int32),
            pltpu.VMEM((EPT,), jnp.int32),
            pltpu.VMEM((EPT,), f32),
            pltpu.VMEM((EPT,), f32),
            pltpu.VMEM((N,), f32),
        ],
    )(_s12_body)
    ex, denom_part = s12(a_src, a_dst, src, dst, ae, bound)

    r, coef_s = pl.pallas_call(
        _k3_body,
        out_shape=[
            jax.ShapeDtypeStruct((N,), f32),
            jax.ShapeDtypeStruct((N,), f32),
        ],
    )(denom_part, bound, alpha_s)

    s3 = functools.partial(
        pl.kernel,
        out_type=[jax.ShapeDtypeStruct((NC, N, HID), f32)],
        mesh=mesh,
        compiler_params=pltpu.CompilerParams(needs_layout_passes=False),
        scratch_types=[
            pltpu.VMEM((N,), f32),
            pltpu.VMEM((3, CH), jnp.int32),
            pltpu.VMEM((3, CH), jnp.int32),
            pltpu.VMEM((CH,), f32),
            pltpu.VMEM((CH,), f32),
            pltpu.VMEM((CH, HID), f32),
            pltpu.VMEM((CH, HID), f32),
            pltpu.VMEM_SHARED((N, HID), f32),
            pltpu.SemaphoreType.DMA,
            pltpu.SemaphoreType.DMA,
            pltpu.SemaphoreType.DMA,
            pltpu.SemaphoreType.DMA,
        ],
    )(_s3_body)
    ex_bits = lax.bitcast_convert_type(ex, jnp.int32)
    edata = jnp.stack([src.reshape(NW, NCH, CH), dst.reshape(NW, NCH, CH),
                       ex_bits.reshape(NW, NCH, CH)], axis=2)
    out_part = s3(edata, r, xs)
    if isinstance(out_part, (tuple, list)):
        out_part = out_part[0]

    out = pl.pallas_call(
        _k4_body,
        out_shape=jax.ShapeDtypeStruct((N, HID), f32),
    )(out_part, xs, coef_s, bias)
    return out


# speedup vs baseline: 18.7367x; 1.0749x over previous
import functools

import jax
import jax.numpy as jnp
from jax import lax
from jax.experimental import pallas as pl
from jax.experimental.pallas import tpu as pltpu
from jax.experimental.pallas import tpu_sc as plsc

HID = 128
N = 10000
E = 320000
NC, NS, L = 2, 16, 16
NW = NC * NS
EPT = E // NW
CH = 80
NCH = EPT // CH
WB = 624
NEG = -1e30


def _k1a_body(x_ref, w_ref, we_ref, att2_ref, atte_ref, xs_ref, a2_ref, v_ref):
    xs = jnp.dot(x_ref[...], w_ref[...], preferred_element_type=jnp.float32)
    xs_ref[...] = xs
    a2_ref[...] = jnp.dot(xs, att2_ref[...], preferred_element_type=jnp.float32)
    v_ref[...] = jnp.dot(we_ref[...], atte_ref[...],
                         preferred_element_type=jnp.float32)


def _k1b_body(ea_ref, v_ref, a2_ref, ae_ref, bound_ref, als_ref, sm_ref):
    i = pl.program_id(0)
    aeb = jnp.dot(ea_ref[...], v_ref[...], preferred_element_type=jnp.float32)
    ae_ref[...] = aeb
    bm = jnp.max(aeb)
    bs = jnp.sum(aeb)

    @pl.when(i == 0)
    def _():
        sm_ref[0] = bm
        sm_ref[1] = bs

    @pl.when(i > 0)
    def _():
        sm_ref[0] = jnp.maximum(sm_ref[0], bm)
        sm_ref[1] = sm_ref[1] + bs

    @pl.when(i == pl.num_programs(0) - 1)
    def _():
        a_src = a2_ref[...][:, 0]
        a_dst = a2_ref[...][:, 1]
        mean = sm_ref[1] * (1.0 / E)
        pre = a_dst + (jnp.max(a_src) + sm_ref[0])
        bound_ref[...] = jnp.where(pre >= 0, pre, 0.2 * pre)
        a = a_src + a_dst + mean
        als_ref[...] = jnp.where(a >= 0, a, 0.2 * a)


def _k3_body(part_ref, bound_ref, als_ref, r_ref, cs_ref):
    ex_s = jnp.exp(als_ref[...] - bound_ref[...])
    denom = jnp.sum(part_ref[...], axis=0) + ex_s
    r = 1.0 / (denom + 1e-16)
    r_ref[...] = r
    cs_ref[...] = ex_s * r


def _k4_body(part_ref, xs_ref, cs_ref, bias_ref, out_ref):
    acc = part_ref[0] + part_ref[1]
    out_ref[...] = acc + cs_ref[...][:, None] * xs_ref[...] + bias_ref[...]


def _leaky(x):
    return jnp.where(x >= 0, x, 0.2 * x)


def _s12_body(asrc_hbm, adst_hbm, src_hbm, dst_hbm, ae_hbm, bound_hbm,
              ex_out, denom_out,
              asrc_v, adst_v, bound_v, src_v, dst_v, ae_v, ex_v, denom_v):
    cid = lax.axis_index("c")
    sid = lax.axis_index("s")
    wid = sid * NC + cid
    base = wid * EPT
    pltpu.sync_copy(asrc_hbm, asrc_v)
    pltpu.sync_copy(adst_hbm, adst_v)
    pltpu.sync_copy(bound_hbm, bound_v)
    pltpu.sync_copy(src_hbm.at[pl.ds(base, EPT)], src_v)
    pltpu.sync_copy(dst_hbm.at[pl.ds(base, EPT)], dst_v)
    pltpu.sync_copy(ae_hbm.at[pl.ds(base, EPT)], ae_v)

    def initb(i, c):
        denom_v[pl.ds(i * L, L)] = jnp.zeros((L,), jnp.float32)
        return c

    lax.fori_loop(0, N // L, initb, 0)

    def edge_block(i, c):
        sl = pl.ds(i * L, L)
        s = src_v[sl]
        d = dst_v[sl]
        a = (plsc.load_gather(asrc_v, [s]) + plsc.load_gather(adst_v, [d])
             + ae_v[sl])
        al = _leaky(a)
        e = jnp.exp(al - plsc.load_gather(bound_v, [d]))
        ex_v[sl] = e
        plsc.addupdate_scatter(denom_v, [d], e)
        return c

    lax.fori_loop(0, EPT // L, edge_block, 0)
    pltpu.sync_copy(ex_v, ex_out.at[pl.ds(base, EPT)])
    pltpu.sync_copy(denom_v, denom_out.at[wid])


def _s3_body(edata_hbm, r_hbm, xs_hbm,
             outp_hbm,
             r_v, ech0_v, ech1_v, coefc0_v, coefc1_v, rows0_v, rows1_v,
             out_sh, gsem0, gsem1, ssem0, ssem1):
    cid = lax.axis_index("c")
    sid = lax.axis_index("s")
    wid = sid * NC + cid
    pltpu.sync_copy(r_hbm, r_v)

    def zrow(i, c):
        for h in range(HID // L):
            rows0_v[i, pl.ds(h * L, L)] = jnp.zeros((L,), jnp.float32)
        return c

    lax.fori_loop(0, CH, zrow, 0)
    for k in range(WB // CH):
        pltpu.sync_copy(rows0_v, out_sh.at[pl.ds(sid * WB + k * CH, CH)])
    pltpu.sync_copy(rows0_v.at[pl.ds(0, WB - (WB // CH) * CH)],
                    out_sh.at[pl.ds(sid * WB + (WB // CH) * CH,
                                    WB - (WB // CH) * CH)])

    @pl.when(sid == 0)
    def _():
        pltpu.sync_copy(rows0_v.at[pl.ds(0, N - NS * WB)],
                        out_sh.at[pl.ds(NS * WB, N - NS * WB)])

    plsc.subcore_barrier()

    def coefs(ech, coefc):
        for j in range(CH // L):
            sl = pl.ds(j * L, L)
            e = plsc.bitcast(ech[2, sl], jnp.float32)
            coefc[sl] = e * plsc.load_gather(r_v, [ech[1, sl]])

    dnums = lax.GatherDimensionNumbers(
        offset_dims=(), collapsed_slice_dims=(0,), start_index_map=(0,))

    def scale(buf, coefc):
        def grp(j, cc):
            c16 = coefc[pl.ds(j * L, L)]
            for w in range(L):
                spl = lax.gather(
                    c16, jnp.full((L, 1), w, jnp.int32), dnums, (1,),
                    mode=lax.GatherScatterMode.PROMISE_IN_BOUNDS)
                row = j * L + w
                for h in range(HID // L):
                    sl2 = pl.ds(h * L, L)
                    buf[row, sl2] = buf[row, sl2] * spl
            return cc

        lax.fori_loop(0, CH // L, grp, 0)

    def pair(i, carry):
        c0 = 2 * i
        c1 = c0 + 1

        @pl.when(i > 0)
        def _():
            pltpu.make_async_copy(rows0_v, out_sh.at[ech0_v.at[1]],
                                  ssem0).wait()
            pltpu.make_async_copy(rows1_v, out_sh.at[ech1_v.at[1]],
                                  ssem1).wait()

        pltpu.sync_copy(edata_hbm.at[wid, c0], ech0_v)
        pltpu.sync_copy(edata_hbm.at[wid, c1], ech1_v)
        g0 = pltpu.async_copy(xs_hbm.at[ech0_v.at[0]], rows0_v, gsem0)
        g1 = pltpu.async_copy(xs_hbm.at[ech1_v.at[0]], rows1_v, gsem1)
        coefs(ech0_v, coefc0_v)
        coefs(ech1_v, coefc1_v)
        g0.wait()
        scale(rows0_v, coefc0_v)
        pltpu.async_copy(rows0_v, out_sh.at[ech0_v.at[1]], ssem0, add=True)
        g1.wait()
        scale(rows1_v, coefc1_v)
        pltpu.async_copy(rows1_v, out_sh.at[ech1_v.at[1]], ssem1, add=True)
        return carry

    lax.fori_loop(0, NCH // 2, pair, 0)
    pltpu.make_async_copy(rows0_v, out_sh.at[ech0_v.at[1]], ssem0).wait()
    pltpu.make_async_copy(rows1_v, out_sh.at[ech1_v.at[1]], ssem1).wait()
    pltpu.sync_copy(edata_hbm.at[wid, NCH - 1], ech0_v)
    g0 = pltpu.async_copy(xs_hbm.at[ech0_v.at[0]], rows0_v, gsem0)
    coefs(ech0_v, coefc0_v)
    g0.wait()
    scale(rows0_v, coefc0_v)
    pltpu.sync_copy(rows0_v, out_sh.at[ech0_v.at[1]], add=True)

    plsc.subcore_barrier()
    pltpu.sync_copy(out_sh.at[pl.ds(sid * WB, WB)],
                    outp_hbm.at[cid, pl.ds(sid * WB, WB)])

    @pl.when(sid == 0)
    def _():
        pltpu.sync_copy(out_sh.at[pl.ds(NS * WB, N - NS * WB)],
                        outp_hbm.at[cid, pl.ds(NS * WB, N - NS * WB)])


def kernel(edge_attr, edge_index, entity_count, node_table, W, att_src,
           att_dst, We, att_edge, bias):
    f32 = jnp.float32
    valid = jnp.arange(N, dtype=jnp.int32) < entity_count
    x = jnp.where(valid[:, None], node_table[:N], 0.0).astype(f32)
    src = edge_index[0]
    dst = edge_index[1]
    att2 = jnp.stack([att_src, att_dst], axis=1)
    atte = att_edge[:, None]

    xs, a2, v2 = pl.pallas_call(
        _k1a_body,
        out_shape=[
            jax.ShapeDtypeStruct((N, HID), f32),
            jax.ShapeDtypeStruct((N, 2), f32),
            jax.ShapeDtypeStruct((HID, 1), f32),
        ],
    )(x, W, We, att2, atte)
    a_src = a2[:, 0]
    a_dst = a2[:, 1]

    EB = 12800
    ae2, bound, alpha_s = pl.pallas_call(
        _k1b_body,
        grid=(E // EB,),
        in_specs=[
            pl.BlockSpec((EB, HID), lambda i: (i, 0)),
            pl.BlockSpec((HID, 1), lambda i: (0, 0)),
            pl.BlockSpec((N, 2), lambda i: (0, 0)),
        ],
        out_specs=[
            pl.BlockSpec((EB, 1), lambda i: (i, 0)),
            pl.BlockSpec((N,), lambda i: (0,)),
            pl.BlockSpec((N,), lambda i: (0,)),
        ],
        out_shape=[
            jax.ShapeDtypeStruct((E, 1), f32),
            jax.ShapeDtypeStruct((N,), f32),
            jax.ShapeDtypeStruct((N,), f32),
        ],
        scratch_shapes=[pltpu.SMEM((2,), f32)],
    )(edge_attr, v2, a2)
    ae = ae2.reshape(E)

    mesh = plsc.VectorSubcoreMesh(core_axis_name="c", subcore_axis_name="s",
                                  num_cores=NC, num_subcores=NS)

    s12 = functools.partial(
        pl.kernel,
        out_type=[
            jax.ShapeDtypeStruct((E,), f32),
            jax.ShapeDtypeStruct((NW, N), f32),
        ],
        mesh=mesh,
        compiler_params=pltpu.CompilerParams(needs_layout_passes=False),
        scratch_types=[
            pltpu.VMEM((N,), f32),
            pltpu.VMEM((N,), f32),
            pltpu.VMEM((N,), f32),
            pltpu.VMEM((EPT,), jnp.int32),
            pltpu.VMEM((EPT,), jnp.int32),
            pltpu.VMEM((EPT,), f32),
            pltpu.VMEM((EPT,), f32),
            pltpu.VMEM((N,), f32),
        ],
    )(_s12_body)
    ex, denom_part = s12(a_src, a_dst, src, dst, ae, bound)

    r, coef_s = pl.pallas_call(
        _k3_body,
        out_shape=[
            jax.ShapeDtypeStruct((N,), f32),
            jax.ShapeDtypeStruct((N,), f32),
        ],
    )(denom_part, bound, alpha_s)

    s3 = functools.partial(
        pl.kernel,
        out_type=[jax.ShapeDtypeStruct((NC, N, HID), f32)],
        mesh=mesh,
        compiler_params=pltpu.CompilerParams(needs_layout_passes=False),
        scratch_types=[
            pltpu.VMEM((N,), f32),
            pltpu.VMEM((3, CH), jnp.int32),
            pltpu.VMEM((3, CH), jnp.int32),
            pltpu.VMEM((CH,), f32),
            pltpu.VMEM((CH,), f32),
            pltpu.VMEM((CH, HID), f32),
            pltpu.VMEM((CH, HID), f32),
            pltpu.VMEM_SHARED((N, HID), f32),
            pltpu.SemaphoreType.DMA,
            pltpu.SemaphoreType.DMA,
            pltpu.SemaphoreType.DMA,
            pltpu.SemaphoreType.DMA,
        ],
    )(_s3_body)
    ex_bits = lax.bitcast_convert_type(ex, jnp.int32)
    edata = jnp.stack([src.reshape(NW, NCH, CH), dst.reshape(NW, NCH, CH),
                       ex_bits.reshape(NW, NCH, CH)], axis=2)
    out_part = s3(edata, r, xs)
    if isinstance(out_part, (tuple, list)):
        out_part = out_part[0]

    out = pl.pallas_call(
        _k4_body,
        out_shape=jax.ShapeDtypeStruct((N, HID), f32),
    )(out_part, xs, coef_s, bias)
    return out


# speedup vs baseline: 24.3736x; 1.3008x over previous
import functools

import jax
import jax.numpy as jnp
from jax import lax
from jax.experimental import pallas as pl
from jax.experimental.pallas import tpu as pltpu
from jax.experimental.pallas import tpu_sc as plsc

HID = 128
N = 10000
E = 320000
NC, NS, L = 2, 16, 16
NW = NC * NS
EPT = E // NW
CH = 80
NCH = EPT // CH
WB = 624
NEG = -1e30


def _k1a_body(x_ref, w_ref, we_ref, att2_ref, atte_ref, xs_ref, a2_ref, v_ref):
    xs = jnp.dot(x_ref[...], w_ref[...], preferred_element_type=jnp.float32)
    xs_ref[...] = xs
    a2_ref[...] = jnp.dot(xs, att2_ref[...], preferred_element_type=jnp.float32)
    v_ref[...] = jnp.dot(we_ref[...], atte_ref[...],
                         preferred_element_type=jnp.float32)


def _k1b_body(ea_ref, v_ref, a2_ref, ae_ref, bound_ref, als_ref, sm_ref):
    i = pl.program_id(0)
    aeb = jnp.dot(ea_ref[...], v_ref[...], preferred_element_type=jnp.float32)
    ae_ref[...] = aeb
    bm = jnp.max(aeb)
    bs = jnp.sum(aeb)

    @pl.when(i == 0)
    def _():
        sm_ref[0] = bm
        sm_ref[1] = bs

    @pl.when(i > 0)
    def _():
        sm_ref[0] = jnp.maximum(sm_ref[0], bm)
        sm_ref[1] = sm_ref[1] + bs

    @pl.when(i == pl.num_programs(0) - 1)
    def _():
        a_src = a2_ref[...][:, 0]
        a_dst = a2_ref[...][:, 1]
        mean = sm_ref[1] * (1.0 / E)
        pre = a_dst + (jnp.max(a_src) + sm_ref[0])
        bound_ref[...] = jnp.where(pre >= 0, pre, 0.2 * pre)
        a = a_src + a_dst + mean
        als_ref[...] = jnp.where(a >= 0, a, 0.2 * a)


def _k3_body(part_ref, bound_ref, als_ref, r_ref, cs_ref):
    ex_s = jnp.exp(als_ref[...] - bound_ref[...])
    denom = jnp.sum(part_ref[...], axis=0) + ex_s
    r = 1.0 / (denom + 1e-16)
    r_ref[...] = r
    cs_ref[...] = ex_s * r


def _k4_body(part_ref, xs_ref, cs_ref, bias_ref, out_ref):
    acc = part_ref[0] + part_ref[1]
    out_ref[...] = acc + cs_ref[...][:, None] * xs_ref[...] + bias_ref[...]


def _leaky(x):
    return jnp.where(x >= 0, x, 0.2 * x)


def _s12_body(asrc_hbm, adst_hbm, src_hbm, dst_hbm, ae_hbm, bound_hbm,
              ex_out, denom_out,
              asrc_v, adst_v, bound_v, src_v, dst_v, ae_v, ex_v, denom_v):
    cid = lax.axis_index("c")
    sid = lax.axis_index("s")
    wid = sid * NC + cid
    base = wid * EPT
    pltpu.sync_copy(asrc_hbm, asrc_v)
    pltpu.sync_copy(adst_hbm, adst_v)
    pltpu.sync_copy(bound_hbm, bound_v)
    pltpu.sync_copy(src_hbm.at[pl.ds(base, EPT)], src_v)
    pltpu.sync_copy(dst_hbm.at[pl.ds(base, EPT)], dst_v)
    pltpu.sync_copy(ae_hbm.at[pl.ds(base, EPT)], ae_v)

    def initb(i, c):
        denom_v[pl.ds(i * L, L)] = jnp.zeros((L,), jnp.float32)
        return c

    lax.fori_loop(0, N // L, initb, 0)

    def edge_block(i, c):
        sl = pl.ds(i * L, L)
        s = src_v[sl]
        d = dst_v[sl]
        a = (plsc.load_gather(asrc_v, [s]) + plsc.load_gather(adst_v, [d])
             + ae_v[sl])
        al = _leaky(a)
        e = jnp.exp(al - plsc.load_gather(bound_v, [d]))
        ex_v[sl] = e
        plsc.addupdate_scatter(denom_v, [d], e)
        return c

    lax.fori_loop(0, EPT // L, edge_block, 0)
    pltpu.sync_copy(ex_v, ex_out.at[pl.ds(base, EPT)])
    pltpu.sync_copy(denom_v, denom_out.at[wid])


def _s3_body(edata_hbm, r_hbm, xs_hbm,
             outp_hbm,
             r_v, ech0_v, ech1_v, ech2_v, coefc0_v, coefc1_v, coefc2_v,
             dsc0_v, dsc1_v, dsc2_v, rows0_v, rows1_v, rows2_v, out_sh,
             gsem0, gsem1, gsem2, esem0, esem1, esem2, ssem0, ssem1, ssem2):
    cid = lax.axis_index("c")
    sid = lax.axis_index("s")
    wid = sid * NC + cid
    pltpu.sync_copy(r_hbm, r_v)

    def zrow(i, c):
        for h in range(HID // L):
            rows0_v[i, pl.ds(h * L, L)] = jnp.zeros((L,), jnp.float32)
        return c

    lax.fori_loop(0, CH, zrow, 0)
    for k in range(WB // CH):
        pltpu.sync_copy(rows0_v, out_sh.at[pl.ds(sid * WB + k * CH, CH)])
    pltpu.sync_copy(rows0_v.at[pl.ds(0, WB - (WB // CH) * CH)],
                    out_sh.at[pl.ds(sid * WB + (WB // CH) * CH,
                                    WB - (WB // CH) * CH)])

    @pl.when(sid == 0)
    def _():
        pltpu.sync_copy(rows0_v.at[pl.ds(0, N - NS * WB)],
                        out_sh.at[pl.ds(NS * WB, N - NS * WB)])

    plsc.subcore_barrier()

    def coefs(ech, coefc):
        for j in range(CH // L):
            sl = pl.ds(j * L, L)
            e = plsc.bitcast(ech[2, sl], jnp.float32)
            coefc[sl] = e * plsc.load_gather(r_v, [ech[1, sl]])

    dnums = lax.GatherDimensionNumbers(
        offset_dims=(), collapsed_slice_dims=(0,), start_index_map=(0,))

    def scale(buf, coefc):
        def grp(j, cc):
            c16 = coefc[pl.ds(j * L, L)]
            for w in range(L):
                spl = lax.gather(
                    c16, jnp.full((L, 1), w, jnp.int32), dnums, (1,),
                    mode=lax.GatherScatterMode.PROMISE_IN_BOUNDS)
                row = j * L + w
                for h in range(HID // L):
                    sl2 = pl.ds(h * L, L)
                    buf[row, sl2] = buf[row, sl2] * spl
            return cc

        lax.fori_loop(0, CH // L, grp, 0)

    ech = (ech0_v, ech1_v, ech2_v)
    cfc = (coefc0_v, coefc1_v, coefc2_v)
    dsc = (dsc0_v, dsc1_v, dsc2_v)
    rows = (rows0_v, rows1_v, rows2_v)
    gsem = (gsem0, gsem1, gsem2)
    esem = (esem0, esem1, esem2)
    ssem = (ssem0, ssem1, ssem2)

    def step(c, b, drain, pref_ech, pref_gather):
        bp = (b + 2) % 3
        pltpu.make_async_copy(xs_hbm.at[ech[b].at[0]], rows[b],
                              gsem[b]).wait()
        coefs(ech[b], cfc[b])
        for j in range(CH // L):
            sl = pl.ds(j * L, L)
            dsc[b][sl] = ech[b][1, sl]
        if pref_ech is True:
            pltpu.async_copy(edata_hbm.at[wid, c + 3], ech[b], esem[b])
        elif pref_ech is not None:
            @pl.when(pref_ech)
            def _():
                pltpu.async_copy(edata_hbm.at[wid, c + 3], ech[b], esem[b])
        scale(rows[b], cfc[b])
        pltpu.async_copy(rows[b], out_sh.at[dsc[b]], ssem[b], add=True)
        if drain is True:
            pltpu.make_async_copy(rows[bp], out_sh.at[dsc[bp]],
                                  ssem[bp]).wait()
        elif drain is not None:
            @pl.when(drain)
            def _():
                pltpu.make_async_copy(rows[bp], out_sh.at[dsc[bp]],
                                      ssem[bp]).wait()
        if pref_gather:
            pltpu.make_async_copy(edata_hbm.at[wid, c + 2], ech[bp],
                                  esem[bp]).wait()
            pltpu.async_copy(xs_hbm.at[ech[bp].at[0]], rows[bp], gsem[bp])

    for b in range(3):
        pltpu.async_copy(edata_hbm.at[wid, b], ech[b], esem[b])
    for b in range(2):
        pltpu.make_async_copy(edata_hbm.at[wid, b], ech[b], esem[b]).wait()
        pltpu.async_copy(xs_hbm.at[ech[b].at[0]], rows[b], gsem[b])

    def triple(i, carry):
        c0 = 3 * i
        step(c0, 0, i > 0, True, True)
        step(c0 + 1, 1, True, True, True)
        step(c0 + 2, 2, True, c0 + 5 < NCH, True)
        return carry

    lax.fori_loop(0, NCH // 3, triple, 0)
    step(NCH - 2, 0, True, None, False)
    step(NCH - 1, 1, True, None, False)
    pltpu.make_async_copy(rows[1], out_sh.at[dsc[1]], ssem[1]).wait()

    plsc.subcore_barrier()
    pltpu.sync_copy(out_sh.at[pl.ds(sid * WB, WB)],
                    outp_hbm.at[cid, pl.ds(sid * WB, WB)])

    @pl.when(sid == 0)
    def _():
        pltpu.sync_copy(out_sh.at[pl.ds(NS * WB, N - NS * WB)],
                        outp_hbm.at[cid, pl.ds(NS * WB, N - NS * WB)])


def kernel(edge_attr, edge_index, entity_count, node_table, W, att_src,
           att_dst, We, att_edge, bias):
    f32 = jnp.float32
    valid = jnp.arange(N, dtype=jnp.int32) < entity_count
    x = jnp.where(valid[:, None], node_table[:N], 0.0).astype(f32)
    src = edge_index[0]
    dst = edge_index[1]
    att2 = jnp.stack([att_src, att_dst], axis=1)
    atte = att_edge[:, None]

    xs, a2, v2 = pl.pallas_call(
        _k1a_body,
        out_shape=[
            jax.ShapeDtypeStruct((N, HID), f32),
            jax.ShapeDtypeStruct((N, 2), f32),
            jax.ShapeDtypeStruct((HID, 1), f32),
        ],
    )(x, W, We, att2, atte)
    a_src = a2[:, 0]
    a_dst = a2[:, 1]

    EB = 12800
    ae2, bound, alpha_s = pl.pallas_call(
        _k1b_body,
        grid=(E // EB,),
        in_specs=[
            pl.BlockSpec((EB, HID), lambda i: (i, 0)),
            pl.BlockSpec((HID, 1), lambda i: (0, 0)),
            pl.BlockSpec((N, 2), lambda i: (0, 0)),
        ],
        out_specs=[
            pl.BlockSpec((EB, 1), lambda i: (i, 0)),
            pl.BlockSpec((N,), lambda i: (0,)),
            pl.BlockSpec((N,), lambda i: (0,)),
        ],
        out_shape=[
            jax.ShapeDtypeStruct((E, 1), f32),
            jax.ShapeDtypeStruct((N,), f32),
            jax.ShapeDtypeStruct((N,), f32),
        ],
        scratch_shapes=[pltpu.SMEM((2,), f32)],
    )(edge_attr, v2, a2)
    ae = ae2.reshape(E)

    mesh = plsc.VectorSubcoreMesh(core_axis_name="c", subcore_axis_name="s",
                                  num_cores=NC, num_subcores=NS)

    s12 = functools.partial(
        pl.kernel,
        out_type=[
            jax.ShapeDtypeStruct((E,), f32),
            jax.ShapeDtypeStruct((NW, N), f32),
        ],
        mesh=mesh,
        compiler_params=pltpu.CompilerParams(needs_layout_passes=False),
        scratch_types=[
            pltpu.VMEM((N,), f32),
            pltpu.VMEM((N,), f32),
            pltpu.VMEM((N,), f32),
            pltpu.VMEM((EPT,), jnp.int32),
            pltpu.VMEM((EPT,), jnp.int32),
            pltpu.VMEM((EPT,), f32),
            pltpu.VMEM((EPT,), f32),
            pltpu.VMEM((N,), f32),
        ],
    )(_s12_body)
    ex, denom_part = s12(a_src, a_dst, src, dst, ae, bound)

    r, coef_s = pl.pallas_call(
        _k3_body,
        out_shape=[
            jax.ShapeDtypeStruct((N,), f32),
            jax.ShapeDtypeStruct((N,), f32),
        ],
    )(denom_part, bound, alpha_s)

    s3 = functools.partial(
        pl.kernel,
        out_type=[jax.ShapeDtypeStruct((NC, N, HID), f32)],
        mesh=mesh,
        compiler_params=pltpu.CompilerParams(needs_layout_passes=False),
        scratch_types=[
            pltpu.VMEM((N,), f32),
            pltpu.VMEM((3, CH), jnp.int32),
            pltpu.VMEM((3, CH), jnp.int32),
            pltpu.VMEM((3, CH), jnp.int32),
            pltpu.VMEM((CH,), f32),
            pltpu.VMEM((CH,), f32),
            pltpu.VMEM((CH,), f32),
            pltpu.VMEM((CH,), jnp.int32),
            pltpu.VMEM((CH,), jnp.int32),
            pltpu.VMEM((CH,), jnp.int32),
            pltpu.VMEM((CH, HID), f32),
            pltpu.VMEM((CH, HID), f32),
            pltpu.VMEM((CH, HID), f32),
            pltpu.VMEM_SHARED((N, HID), f32),
        ] + [pltpu.SemaphoreType.DMA] * 9,
    )(_s3_body)
    ex_bits = lax.bitcast_convert_type(ex, jnp.int32)
    edata = jnp.stack([src.reshape(NW, NCH, CH), dst.reshape(NW, NCH, CH),
                       ex_bits.reshape(NW, NCH, CH)], axis=2)
    out_part = s3(edata, r, xs)
    if isinstance(out_part, (tuple, list)):
        out_part = out_part[0]

    out = pl.pallas_call(
        _k4_body,
        out_shape=jax.ShapeDtypeStruct((N, HID), f32),
    )(out_part, xs, coef_s, bias)
    return out


# speedup vs baseline: 25.8024x; 1.0586x over previous
import functools

import jax
import jax.numpy as jnp
from jax import lax
from jax.experimental import pallas as pl
from jax.experimental.pallas import tpu as pltpu
from jax.experimental.pallas import tpu_sc as plsc

HID = 128
N = 10000
E = 320000
NC, NS, L = 2, 16, 16
NW = NC * NS
EPT = E // NW
CH = 80
NCH = EPT // CH
WB = 624
_K1G = 25
NEG = -1e30


def _k1_body(nt_ref, ea_ref, ec_ref, w_ref, we_ref, att2_ref, atte_ref,
             xs_ref, a2_ref, ae_ref, bound_ref, als_ref,
             sm_ref, v_ref, a2f_ref):
    i = pl.program_id(0)
    nb = N // _K1G

    @pl.when(i == 0)
    def _():
        v_ref[...] = jnp.dot(we_ref[...], atte_ref[...],
                             preferred_element_type=jnp.float32)

    rid = lax.broadcasted_iota(jnp.int32, (nb, 1), 0) + i * nb
    x = jnp.where(rid < ec_ref[0, 0], nt_ref[...], 0.0)
    xs = jnp.dot(x, w_ref[...], preferred_element_type=jnp.float32)
    xs_ref[...] = xs
    a2 = jnp.dot(xs, att2_ref[...], preferred_element_type=jnp.float32)
    a2_ref[...] = a2
    a2f_ref[pl.ds(i * nb, nb), :] = a2
    ae = jnp.dot(ea_ref[...], v_ref[...], preferred_element_type=jnp.float32)
    ae_ref[...] = ae
    bm = jnp.max(ae)
    bs = jnp.sum(ae)
    bma = jnp.max(a2[:, 0])

    @pl.when(i == 0)
    def _():
        sm_ref[0] = bm
        sm_ref[1] = bs
        sm_ref[2] = bma

    @pl.when(i > 0)
    def _():
        sm_ref[0] = jnp.maximum(sm_ref[0], bm)
        sm_ref[1] = sm_ref[1] + bs
        sm_ref[2] = jnp.maximum(sm_ref[2], bma)

    @pl.when(i == pl.num_programs(0) - 1)
    def _():
        a_src = a2f_ref[...][:, 0]
        a_dst = a2f_ref[...][:, 1]
        mean = sm_ref[1] * (1.0 / E)
        pre = a_dst + (sm_ref[2] + sm_ref[0])
        bound_ref[...] = jnp.where(pre >= 0, pre, 0.2 * pre)
        a = a_src + a_dst + mean
        als_ref[...] = jnp.where(a >= 0, a, 0.2 * a)


def _k3_body(part_ref, bound_ref, als_ref, r_ref, cs_ref):
    ex_s = jnp.exp(als_ref[...] - bound_ref[...])
    denom = jnp.sum(part_ref[...], axis=0) + ex_s
    r = 1.0 / (denom + 1e-16)
    r_ref[...] = r
    cs_ref[...] = ex_s * r


def _k4_body(part_ref, xs_ref, cs_ref, bias_ref, out_ref):
    acc = part_ref[0] + part_ref[1]
    out_ref[...] = acc + cs_ref[...][:, None] * xs_ref[...] + bias_ref[...]


def _leaky(x):
    return jnp.where(x >= 0, x, 0.2 * x)


def _s12_body(a2f_hbm, src_hbm, dst_hbm, ae_hbm, bound_hbm,
              ex_out, denom_out,
              a2f_v, bound_v, src_v, dst_v, ae_v, ex_v, denom_v):
    cid = lax.axis_index("c")
    sid = lax.axis_index("s")
    wid = sid * NC + cid
    base = wid * EPT
    pltpu.sync_copy(a2f_hbm, a2f_v)
    pltpu.sync_copy(bound_hbm, bound_v)
    pltpu.sync_copy(src_hbm.at[pl.ds(base, EPT)], src_v)
    pltpu.sync_copy(dst_hbm.at[pl.ds(base, EPT)], dst_v)
    pltpu.sync_copy(ae_hbm.at[pl.ds(base, EPT)], ae_v)

    def initb(i, c):
        denom_v[pl.ds(i * L, L)] = jnp.zeros((L,), jnp.float32)
        return c

    lax.fori_loop(0, N // L, initb, 0)

    def edge_block(i, c):
        sl = pl.ds(i * L, L)
        s = src_v[sl]
        d = dst_v[sl]
        a = (plsc.load_gather(a2f_v, [s + s])
             + plsc.load_gather(a2f_v, [d + d + 1]) + ae_v[sl])
        al = _leaky(a)
        e = jnp.exp(al - plsc.load_gather(bound_v, [d]))
        ex_v[sl] = e
        plsc.addupdate_scatter(denom_v, [d], e)
        return c

    lax.fori_loop(0, EPT // L, edge_block, 0)
    pltpu.sync_copy(ex_v, ex_out.at[pl.ds(base, EPT)])
    pltpu.sync_copy(denom_v, denom_out.at[wid])


def _s3_body(edata_hbm, r_hbm, xs_hbm,
             outp_hbm,
             r_v, ech0_v, ech1_v, ech2_v, coefc0_v, coefc1_v, coefc2_v,
             dsc0_v, dsc1_v, dsc2_v, rows0_v, rows1_v, rows2_v, out_sh,
             gsem0, gsem1, gsem2, esem0, esem1, esem2, ssem0, ssem1, ssem2):
    cid = lax.axis_index("c")
    sid = lax.axis_index("s")
    wid = sid * NC + cid
    pltpu.sync_copy(r_hbm, r_v)

    def zrow(i, c):
        for h in range(HID // L):
            rows0_v[i, pl.ds(h * L, L)] = jnp.zeros((L,), jnp.float32)
        return c

    lax.fori_loop(0, CH, zrow, 0)
    for k in range(WB // CH):
        pltpu.sync_copy(rows0_v, out_sh.at[pl.ds(sid * WB + k * CH, CH)])
    pltpu.sync_copy(rows0_v.at[pl.ds(0, WB - (WB // CH) * CH)],
                    out_sh.at[pl.ds(sid * WB + (WB // CH) * CH,
                                    WB - (WB // CH) * CH)])

    @pl.when(sid == 0)
    def _():
        pltpu.sync_copy(rows0_v.at[pl.ds(0, N - NS * WB)],
                        out_sh.at[pl.ds(NS * WB, N - NS * WB)])

    plsc.subcore_barrier()

    def coefs(ech, coefc):
        for j in range(CH // L):
            sl = pl.ds(j * L, L)
            e = plsc.bitcast(ech[2, sl], jnp.float32)
            coefc[sl] = e * plsc.load_gather(r_v, [ech[1, sl]])

    dnums = lax.GatherDimensionNumbers(
        offset_dims=(), collapsed_slice_dims=(0,), start_index_map=(0,))

    def scale(buf, coefc):
        def grp(j, cc):
            c16 = coefc[pl.ds(j * L, L)]
            for w in range(L):
                spl = lax.gather(
                    c16, jnp.full((L, 1), w, jnp.int32), dnums, (1,),
                    mode=lax.GatherScatterMode.PROMISE_IN_BOUNDS)
                row = j * L + w
                for h in range(HID // L):
                    sl2 = pl.ds(h * L, L)
                    buf[row, sl2] = buf[row, sl2] * spl
            return cc

        lax.fori_loop(0, CH // L, grp, 0)

    ech = (ech0_v, ech1_v, ech2_v)
    cfc = (coefc0_v, coefc1_v, coefc2_v)
    dsc = (dsc0_v, dsc1_v, dsc2_v)
    rows = (rows0_v, rows1_v, rows2_v)
    gsem = (gsem0, gsem1, gsem2)
    esem = (esem0, esem1, esem2)
    ssem = (ssem0, ssem1, ssem2)

    def step(c, b, drain, pref_ech, pref_gather):
        bp = (b + 2) % 3
        pltpu.make_async_copy(xs_hbm.at[ech[b].at[0]], rows[b],
                              gsem[b]).wait()
        coefs(ech[b], cfc[b])
        for j in range(CH // L):
            sl = pl.ds(j * L, L)
            dsc[b][sl] = ech[b][1, sl]
        if pref_ech is True:
            pltpu.async_copy(edata_hbm.at[wid, c + 3], ech[b], esem[b])
        elif pref_ech is not None:
            @pl.when(pref_ech)
            def _():
                pltpu.async_copy(edata_hbm.at[wid, c + 3], ech[b], esem[b])
        scale(rows[b], cfc[b])
        pltpu.async_copy(rows[b], out_sh.at[dsc[b]], ssem[b], add=True)
        if drain is True:
            pltpu.make_async_copy(rows[bp], out_sh.at[dsc[bp]],
                                  ssem[bp]).wait()
        elif drain is not None:
            @pl.when(drain)
            def _():
                pltpu.make_async_copy(rows[bp], out_sh.at[dsc[bp]],
                                      ssem[bp]).wait()
        if pref_gather:
            pltpu.make_async_copy(edata_hbm.at[wid, c + 2], ech[bp],
                                  esem[bp]).wait()
            pltpu.async_copy(xs_hbm.at[ech[bp].at[0]], rows[bp], gsem[bp])

    for b in range(3):
        pltpu.async_copy(edata_hbm.at[wid, b], ech[b], esem[b])
    for b in range(2):
        pltpu.make_async_copy(edata_hbm.at[wid, b], ech[b], esem[b]).wait()
        pltpu.async_copy(xs_hbm.at[ech[b].at[0]], rows[b], gsem[b])

    def triple(i, carry):
        c0 = 3 * i
        step(c0, 0, i > 0, True, True)
        step(c0 + 1, 1, True, True, True)
        step(c0 + 2, 2, True, c0 + 5 < NCH, True)
        return carry

    lax.fori_loop(0, NCH // 3, triple, 0)
    step(NCH - 2, 0, True, None, False)
    step(NCH - 1, 1, True, None, False)
    pltpu.make_async_copy(rows[1], out_sh.at[dsc[1]], ssem[1]).wait()

    plsc.subcore_barrier()
    pltpu.sync_copy(out_sh.at[pl.ds(sid * WB, WB)],
                    outp_hbm.at[cid, pl.ds(sid * WB, WB)])

    @pl.when(sid == 0)
    def _():
        pltpu.sync_copy(out_sh.at[pl.ds(NS * WB, N - NS * WB)],
                        outp_hbm.at[cid, pl.ds(NS * WB, N - NS * WB)])


def kernel(edge_attr, edge_index, entity_count, node_table, W, att_src,
           att_dst, We, att_edge, bias):
    f32 = jnp.float32
    src = edge_index[0]
    dst = edge_index[1]
    att2 = jnp.stack([att_src, att_dst], axis=1)
    atte = att_edge[:, None]
    ec = jnp.asarray(entity_count, jnp.int32).reshape(1, 1)

    EB = E // _K1G
    NB = N // _K1G
    xs, a2, ae2, bound, alpha_s = pl.pallas_call(
        _k1_body,
        grid=(_K1G,),
        in_specs=[
            pl.BlockSpec((NB, HID), lambda i: (i, 0)),
            pl.BlockSpec((EB, HID), lambda i: (i, 0)),
            pl.BlockSpec(memory_space=pltpu.SMEM),
            pl.BlockSpec((HID, HID), lambda i: (0, 0)),
            pl.BlockSpec((HID, HID), lambda i: (0, 0)),
            pl.BlockSpec((HID, 2), lambda i: (0, 0)),
            pl.BlockSpec((HID, 1), lambda i: (0, 0)),
        ],
        out_specs=[
            pl.BlockSpec((NB, HID), lambda i: (i, 0)),
            pl.BlockSpec((NB, 2), lambda i: (i, 0)),
            pl.BlockSpec((EB, 1), lambda i: (i, 0)),
            pl.BlockSpec((N,), lambda i: (0,)),
            pl.BlockSpec((N,), lambda i: (0,)),
        ],
        out_shape=[
            jax.ShapeDtypeStruct((N, HID), f32),
            jax.ShapeDtypeStruct((N, 2), f32),
            jax.ShapeDtypeStruct((E, 1), f32),
            jax.ShapeDtypeStruct((N,), f32),
            jax.ShapeDtypeStruct((N,), f32),
        ],
        scratch_shapes=[pltpu.SMEM((4,), f32), pltpu.VMEM((HID, 1), f32),
                        pltpu.VMEM((N, 2), f32)],
    )(node_table, edge_attr, ec, W, We, att2, atte)
    ae = ae2.reshape(E)

    mesh = plsc.VectorSubcoreMesh(core_axis_name="c", subcore_axis_name="s",
                                  num_cores=NC, num_subcores=NS)

    s12 = functools.partial(
        pl.kernel,
        out_type=[
            jax.ShapeDtypeStruct((E,), f32),
            jax.ShapeDtypeStruct((NW, N), f32),
        ],
        mesh=mesh,
        compiler_params=pltpu.CompilerParams(needs_layout_passes=False),
        scratch_types=[
            pltpu.VMEM((2 * N,), f32),
            pltpu.VMEM((N,), f32),
            pltpu.VMEM((EPT,), jnp.int32),
            pltpu.VMEM((EPT,), jnp.int32),
            pltpu.VMEM((EPT,), f32),
            pltpu.VMEM((EPT,), f32),
            pltpu.VMEM((N,), f32),
        ],
    )(_s12_body)
    ex, denom_part = s12(a2.reshape(2 * N), src, dst, ae, bound)

    r, coef_s = pl.pallas_call(
        _k3_body,
        out_shape=[
            jax.ShapeDtypeStruct((N,), f32),
            jax.ShapeDtypeStruct((N,), f32),
        ],
    )(denom_part, bound, alpha_s)

    s3 = functools.partial(
        pl.kernel,
        out_type=[jax.ShapeDtypeStruct((NC, N, HID), f32)],
        mesh=mesh,
        compiler_params=pltpu.CompilerParams(needs_layout_passes=False),
        scratch_types=[
            pltpu.VMEM((N,), f32),
            pltpu.VMEM((3, CH), jnp.int32),
            pltpu.VMEM((3, CH), jnp.int32),
            pltpu.VMEM((3, CH), jnp.int32),
            pltpu.VMEM((CH,), f32),
            pltpu.VMEM((CH,), f32),
            pltpu.VMEM((CH,), f32),
            pltpu.VMEM((CH,), jnp.int32),
            pltpu.VMEM((CH,), jnp.int32),
            pltpu.VMEM((CH,), jnp.int32),
            pltpu.VMEM((CH, HID), f32),
            pltpu.VMEM((CH, HID), f32),
            pltpu.VMEM((CH, HID), f32),
            pltpu.VMEM_SHARED((N, HID), f32),
        ] + [pltpu.SemaphoreType.DMA] * 9,
    )(_s3_body)
    ex_bits = lax.bitcast_convert_type(ex, jnp.int32)
    edata = jnp.stack([src.reshape(NW, NCH, CH), dst.reshape(NW, NCH, CH),
                       ex_bits.reshape(NW, NCH, CH)], axis=2)
    out_part = s3(edata, r, xs)
    if isinstance(out_part, (tuple, list)):
        out_part = out_part[0]

    out = pl.pallas_call(
        _k4_body,
        out_shape=jax.ShapeDtypeStruct((N, HID), f32),
    )(out_part, xs, coef_s, bias)
    return out


# speedup vs baseline: 26.3707x; 1.0220x over previous
import functools

import jax
import jax.numpy as jnp
from jax import lax
from jax.experimental import pallas as pl
from jax.experimental.pallas import tpu as pltpu
from jax.experimental.pallas import tpu_sc as plsc

HID = 128
N = 10000
E = 320000
NC, NS, L = 2, 16, 16
NW = NC * NS
EPT = E // NW
CH = 80
NCH = EPT // CH
WB = 624
_K1G = 25
NEG = -1e30


def _k1_body(nt_ref, ea_ref, ec_ref, w_ref, we_ref, att2_ref, atte_ref,
             xs_ref, a2_ref, ae_ref, bound_ref, als_ref,
             sm_ref, v_ref, a2f_ref):
    i = pl.program_id(0)
    nb = N // _K1G

    @pl.when(i == 0)
    def _():
        v_ref[...] = jnp.dot(we_ref[...], atte_ref[...],
                             preferred_element_type=jnp.float32).T

    rid = lax.broadcasted_iota(jnp.int32, (nb, 1), 0) + i * nb
    x = jnp.where(rid < ec_ref[0, 0], nt_ref[...], 0.0)
    xs = jnp.dot(x, w_ref[...], preferred_element_type=jnp.float32)
    xs_ref[...] = xs
    a2 = jnp.dot(xs, att2_ref[...], preferred_element_type=jnp.float32)
    a2f_ref[pl.ds(i * nb, nb), :] = a2
    ae = jnp.sum(ea_ref[...] * v_ref[...], axis=1)
    ae_ref[pl.ds(i * (E // _K1G), E // _K1G)] = ae
    bm = jnp.max(ae)
    bs = jnp.sum(ae)
    bma = jnp.max(a2[:, 0])

    @pl.when(i == 0)
    def _():
        sm_ref[0] = bm
        sm_ref[1] = bs
        sm_ref[2] = bma

    @pl.when(i > 0)
    def _():
        sm_ref[0] = jnp.maximum(sm_ref[0], bm)
        sm_ref[1] = sm_ref[1] + bs
        sm_ref[2] = jnp.maximum(sm_ref[2], bma)

    @pl.when(i == pl.num_programs(0) - 1)
    def _():
        a_src = a2f_ref[...][:, 0]
        a_dst = a2f_ref[...][:, 1]
        mean = sm_ref[1] * (1.0 / E)
        pre = a_dst + (sm_ref[2] + sm_ref[0])
        bound_ref[...] = jnp.where(pre >= 0, pre, 0.2 * pre)
        a = a_src + a_dst + mean
        als_ref[...] = jnp.where(a >= 0, a, 0.2 * a)
        a2_ref[...] = a2f_ref[...].T


def _k3_body(part_ref, bound_ref, als_ref, r_ref, cs_ref):
    ex_s = jnp.exp(als_ref[...] - bound_ref[...])
    denom = jnp.sum(part_ref[...], axis=0) + ex_s
    r = 1.0 / (denom + 1e-16)
    r_ref[...] = r
    cs_ref[...] = ex_s * r


def _k4_body(part_ref, xs_ref, cs_ref, bias_ref, out_ref):
    acc = part_ref[0] + part_ref[1]
    out_ref[...] = acc + cs_ref[...][:, None] * xs_ref[...] + bias_ref[...]


def _leaky(x):
    return jnp.where(x >= 0, x, 0.2 * x)


def _s12_body(a2t_hbm, src_hbm, dst_hbm, ae_hbm, bound_hbm,
              ex_out, denom_out,
              asrc_v, adst_v, bound_v, src_v, dst_v, ae_v, ex_v, denom_v):
    cid = lax.axis_index("c")
    sid = lax.axis_index("s")
    wid = sid * NC + cid
    base = wid * EPT
    pltpu.sync_copy(a2t_hbm.at[0], asrc_v)
    pltpu.sync_copy(a2t_hbm.at[1], adst_v)
    pltpu.sync_copy(bound_hbm, bound_v)
    pltpu.sync_copy(src_hbm.at[pl.ds(base, EPT)], src_v)
    pltpu.sync_copy(dst_hbm.at[pl.ds(base, EPT)], dst_v)
    pltpu.sync_copy(ae_hbm.at[pl.ds(base, EPT)], ae_v)

    def initb(i, c):
        denom_v[pl.ds(i * L, L)] = jnp.zeros((L,), jnp.float32)
        return c

    lax.fori_loop(0, N // L, initb, 0)

    def edge_block(i, c):
        sl = pl.ds(i * L, L)
        s = src_v[sl]
        d = dst_v[sl]
        a = (plsc.load_gather(asrc_v, [s]) + plsc.load_gather(adst_v, [d])
             + ae_v[sl])
        al = _leaky(a)
        e = jnp.exp(al - plsc.load_gather(bound_v, [d]))
        ex_v[sl] = e
        plsc.addupdate_scatter(denom_v, [d], e)
        return c

    lax.fori_loop(0, EPT // L, edge_block, 0)
    pltpu.sync_copy(ex_v, ex_out.at[pl.ds(base, EPT)])
    pltpu.sync_copy(denom_v, denom_out.at[wid])


def _s3_body(edata_hbm, r_hbm, xs_hbm,
             outp_hbm,
             r_v, ech0_v, ech1_v, ech2_v, coefc0_v, coefc1_v, coefc2_v,
             dsc0_v, dsc1_v, dsc2_v, rows0_v, rows1_v, rows2_v, out_sh,
             gsem0, gsem1, gsem2, esem0, esem1, esem2, ssem0, ssem1, ssem2):
    cid = lax.axis_index("c")
    sid = lax.axis_index("s")
    wid = sid * NC + cid
    pltpu.sync_copy(r_hbm, r_v)

    def zrow(i, c):
        for h in range(HID // L):
            rows0_v[i, pl.ds(h * L, L)] = jnp.zeros((L,), jnp.float32)
        return c

    lax.fori_loop(0, CH, zrow, 0)
    for k in range(WB // CH):
        pltpu.sync_copy(rows0_v, out_sh.at[pl.ds(sid * WB + k * CH, CH)])
    pltpu.sync_copy(rows0_v.at[pl.ds(0, WB - (WB // CH) * CH)],
                    out_sh.at[pl.ds(sid * WB + (WB // CH) * CH,
                                    WB - (WB // CH) * CH)])

    @pl.when(sid == 0)
    def _():
        pltpu.sync_copy(rows0_v.at[pl.ds(0, N - NS * WB)],
                        out_sh.at[pl.ds(NS * WB, N - NS * WB)])

    plsc.subcore_barrier()

    def coefs(ech, coefc):
        for j in range(CH // L):
            sl = pl.ds(j * L, L)
            e = plsc.bitcast(ech[2, sl], jnp.float32)
            coefc[sl] = e * plsc.load_gather(r_v, [ech[1, sl]])

    dnums = lax.GatherDimensionNumbers(
        offset_dims=(), collapsed_slice_dims=(0,), start_index_map=(0,))

    def scale(buf, coefc):
        def grp(j, cc):
            c16 = coefc[pl.ds(j * L, L)]
            for w in range(L):
                spl = lax.gather(
                    c16, jnp.full((L, 1), w, jnp.int32), dnums, (1,),
                    mode=lax.GatherScatterMode.PROMISE_IN_BOUNDS)
                row = j * L + w
                for h in range(HID // L):
                    sl2 = pl.ds(h * L, L)
                    buf[row, sl2] = buf[row, sl2] * spl
            return cc

        lax.fori_loop(0, CH // L, grp, 0)

    ech = (ech0_v, ech1_v, ech2_v)
    cfc = (coefc0_v, coefc1_v, coefc2_v)
    dsc = (dsc0_v, dsc1_v, dsc2_v)
    rows = (rows0_v, rows1_v, rows2_v)
    gsem = (gsem0, gsem1, gsem2)
    esem = (esem0, esem1, esem2)
    ssem = (ssem0, ssem1, ssem2)

    def step(c, b, drain, pref_ech, pref_gather):
        bp = (b + 2) % 3
        pltpu.make_async_copy(xs_hbm.at[ech[b].at[0]], rows[b],
                              gsem[b]).wait()
        coefs(ech[b], cfc[b])
        for j in range(CH // L):
            sl = pl.ds(j * L, L)
            dsc[b][sl] = ech[b][1, sl]
        if pref_ech is True:
            pltpu.async_copy(edata_hbm.at[wid, c + 3], ech[b], esem[b])
        elif pref_ech is not None:
            @pl.when(pref_ech)
            def _():
                pltpu.async_copy(edata_hbm.at[wid, c + 3], ech[b], esem[b])
        scale(rows[b], cfc[b])
        pltpu.async_copy(rows[b], out_sh.at[dsc[b]], ssem[b], add=True)
        if drain is True:
            pltpu.make_async_copy(rows[bp], out_sh.at[dsc[bp]],
                                  ssem[bp]).wait()
        elif drain is not None:
            @pl.when(drain)
            def _():
                pltpu.make_async_copy(rows[bp], out_sh.at[dsc[bp]],
                                      ssem[bp]).wait()
        if pref_gather:
            pltpu.make_async_copy(edata_hbm.at[wid, c + 2], ech[bp],
                                  esem[bp]).wait()
            pltpu.async_copy(xs_hbm.at[ech[bp].at[0]], rows[bp], gsem[bp])

    for b in range(3):
        pltpu.async_copy(edata_hbm.at[wid, b], ech[b], esem[b])
    for b in range(2):
        pltpu.make_async_copy(edata_hbm.at[wid, b], ech[b], esem[b]).wait()
        pltpu.async_copy(xs_hbm.at[ech[b].at[0]], rows[b], gsem[b])

    def triple(i, carry):
        c0 = 3 * i
        step(c0, 0, i > 0, True, True)
        step(c0 + 1, 1, True, True, True)
        step(c0 + 2, 2, True, c0 + 5 < NCH, True)
        return carry

    lax.fori_loop(0, NCH // 3, triple, 0)
    step(NCH - 2, 0, True, None, False)
    step(NCH - 1, 1, True, None, False)
    pltpu.make_async_copy(rows[1], out_sh.at[dsc[1]], ssem[1]).wait()

    plsc.subcore_barrier()
    pltpu.sync_copy(out_sh.at[pl.ds(sid * WB, WB)],
                    outp_hbm.at[cid, pl.ds(sid * WB, WB)])

    @pl.when(sid == 0)
    def _():
        pltpu.sync_copy(out_sh.at[pl.ds(NS * WB, N - NS * WB)],
                        outp_hbm.at[cid, pl.ds(NS * WB, N - NS * WB)])


def kernel(edge_attr, edge_index, entity_count, node_table, W, att_src,
           att_dst, We, att_edge, bias):
    f32 = jnp.float32
    src = edge_index[0]
    dst = edge_index[1]
    att2 = jnp.stack([att_src, att_dst], axis=1)
    atte = att_edge[:, None]
    ec = jnp.asarray(entity_count, jnp.int32).reshape(1, 1)

    EB = E // _K1G
    NB = N // _K1G
    xs, a2, ae2, bound, alpha_s = pl.pallas_call(
        _k1_body,
        grid=(_K1G,),
        in_specs=[
            pl.BlockSpec((NB, HID), lambda i: (i, 0)),
            pl.BlockSpec((EB, HID), lambda i: (i, 0)),
            pl.BlockSpec(memory_space=pltpu.SMEM),
            pl.BlockSpec((HID, HID), lambda i: (0, 0)),
            pl.BlockSpec((HID, HID), lambda i: (0, 0)),
            pl.BlockSpec((HID, 2), lambda i: (0, 0)),
            pl.BlockSpec((HID, 1), lambda i: (0, 0)),
        ],
        out_specs=[
            pl.BlockSpec((NB, HID), lambda i: (i, 0)),
            pl.BlockSpec((2, N), lambda i: (0, 0)),
            pl.BlockSpec((E,), lambda i: (0,)),
            pl.BlockSpec((N,), lambda i: (0,)),
            pl.BlockSpec((N,), lambda i: (0,)),
        ],
        out_shape=[
            jax.ShapeDtypeStruct((N, HID), f32),
            jax.ShapeDtypeStruct((2, N), f32),
            jax.ShapeDtypeStruct((E,), f32),
            jax.ShapeDtypeStruct((N,), f32),
            jax.ShapeDtypeStruct((N,), f32),
        ],
        scratch_shapes=[pltpu.SMEM((4,), f32), pltpu.VMEM((1, HID), f32),
                        pltpu.VMEM((N, 2), f32)],
    )(node_table, edge_attr, ec, W, We, att2, atte)
    ae = ae2

    mesh = plsc.VectorSubcoreMesh(core_axis_name="c", subcore_axis_name="s",
                                  num_cores=NC, num_subcores=NS)

    s12 = functools.partial(
        pl.kernel,
        out_type=[
            jax.ShapeDtypeStruct((E,), f32),
            jax.ShapeDtypeStruct((NW, N), f32),
        ],
        mesh=mesh,
        compiler_params=pltpu.CompilerParams(needs_layout_passes=False),
        scratch_types=[
            pltpu.VMEM((N,), f32),
            pltpu.VMEM((N,), f32),
            pltpu.VMEM((N,), f32),
            pltpu.VMEM((EPT,), jnp.int32),
            pltpu.VMEM((EPT,), jnp.int32),
            pltpu.VMEM((EPT,), f32),
            pltpu.VMEM((EPT,), f32),
            pltpu.VMEM((N,), f32),
        ],
    )(_s12_body)
    ex, denom_part = s12(a2, src, dst, ae, bound)

    r, coef_s = pl.pallas_call(
        _k3_body,
        out_shape=[
            jax.ShapeDtypeStruct((N,), f32),
            jax.ShapeDtypeStruct((N,), f32),
        ],
    )(denom_part, bound, alpha_s)

    s3 = functools.partial(
        pl.kernel,
        out_type=[jax.ShapeDtypeStruct((NC, N, HID), f32)],
        mesh=mesh,
        compiler_params=pltpu.CompilerParams(needs_layout_passes=False),
        scratch_types=[
            pltpu.VMEM((N,), f32),
            pltpu.VMEM((3, CH), jnp.int32),
            pltpu.VMEM((3, CH), jnp.int32),
            pltpu.VMEM((3, CH), jnp.int32),
            pltpu.VMEM((CH,), f32),
            pltpu.VMEM((CH,), f32),
            pltpu.VMEM((CH,), f32),
            pltpu.VMEM((CH,), jnp.int32),
            pltpu.VMEM((CH,), jnp.int32),
            pltpu.VMEM((CH,), jnp.int32),
            pltpu.VMEM((CH, HID), f32),
            pltpu.VMEM((CH, HID), f32),
            pltpu.VMEM((CH, HID), f32),
            pltpu.VMEM_SHARED((N, HID), f32),
        ] + [pltpu.SemaphoreType.DMA] * 9,
    )(_s3_body)
    ex_bits = lax.bitcast_convert_type(ex, jnp.int32)
    edata = jnp.stack([src.reshape(NW, NCH, CH), dst.reshape(NW, NCH, CH),
                       ex_bits.reshape(NW, NCH, CH)], axis=2)
    out_part = s3(edata, r, xs)
    if isinstance(out_part, (tuple, list)):
        out_part = out_part[0]

    out = pl.pallas_call(
        _k4_body,
        out_shape=jax.ShapeDtypeStruct((N, HID), f32),
    )(out_part, xs, coef_s, bias)
    return out
